# Initial kernel scaffold; baseline (speedup 1.0000x reference)
import jax
import jax.numpy as jnp
from jax.experimental import pallas as pl


def kernel(x, edge_index, edge_attr, W1, b1, root, bconv, gamma, beta, Wlin, blin, Wq1, bq1, Wq2, bq2):
    raise NotImplementedError("write your pallas kernel here")


# speedup vs baseline: 1.2709x; 1.2709x over previous
import functools

import jax
import jax.numpy as jnp
from jax import lax
from jax.experimental import pallas as pl
from jax.experimental.pallas import tpu as pltpu
from jax.experimental.pallas import tpu_sc as plsc

NC = 2
NS = 16
NW = NC * NS
CH = 128
MSGW = 16


def kernel(x, edge_index, edge_attr, W1, b1, root, bconv, gamma, beta,
           Wlin, blin, Wq1, bq1, Wq2, bq2):
    N, IN = x.shape
    E, EDIM = edge_attr.shape
    HID = root.shape[1]
    OUT = Wlin.shape[0]
    HDIM = Wq1.shape[0]
    NA = Wq2.shape[0]

    src = edge_index[0]
    dst = edge_index[1]

    KP = 24
    W1q = W1.reshape(IN, HID, EDIM).transpose(2, 1, 0).reshape(EDIM, HID * IN)
    b1q = b1.reshape(IN, HID).T.reshape(1, HID * IN)
    Wk = jnp.zeros((KP, HID * IN), jnp.float32)
    Wk = Wk.at[:EDIM].set(W1q).at[EDIM:EDIM + 1].set(b1q)
    ea_aug = jnp.concatenate(
        [edge_attr, jnp.ones((E, 1), jnp.float32),
         jnp.zeros((E, KP - EDIM - 1), jnp.float32)], axis=1)

    EPW = E // NW
    NCHUNK = EPW // CH
    TAIL = EPW - NCHUNK * CH

    mesh = plsc.VectorSubcoreMesh(core_axis_name="c", subcore_axis_name="s",
                                  num_cores=NC, num_subcores=NS)

    @functools.partial(
        pl.kernel,
        out_type=jax.ShapeDtypeStruct((E, IN), jnp.float32),
        mesh=mesh,
        scratch_types=[pltpu.VMEM((CH,), jnp.int32),
                       pltpu.VMEM((CH, IN), jnp.float32),
                       pltpu.VMEM((max(TAIL, 8),), jnp.int32),
                       pltpu.VMEM((max(TAIL, 8), IN), jnp.float32),
                       pltpu.SemaphoreType.DMA],
    )
    def gather_k(x_hbm, src_hbm, out_hbm, idx_v, rows_v, idx_t, rows_t, sem):
        wid = lax.axis_index("c") * NS + lax.axis_index("s")
        base = wid * EPW

        def body(i, c):
            off = base + i * CH
            pltpu.sync_copy(src_hbm.at[pl.ds(off, CH)], idx_v)
            pltpu.async_copy(x_hbm.at[idx_v], rows_v, sem).wait()
            pltpu.sync_copy(rows_v, out_hbm.at[pl.ds(off, CH)])
            return c

        lax.fori_loop(0, NCHUNK, body, 0)
        if TAIL:
            off = base + NCHUNK * CH
            pltpu.sync_copy(src_hbm.at[pl.ds(off, TAIL)], idx_t)
            pltpu.async_copy(x_hbm.at[idx_t], rows_t, sem).wait()
            pltpu.sync_copy(rows_t, out_hbm.at[pl.ds(off, TAIL)])

    x_src = gather_k(x, src)

    TE = 640
    GE = E // TE

    def edge_body(ea_ref, xs_ref, w_ref, out_ref):
        P = jnp.dot(ea_ref[...], w_ref[...],
                    preferred_element_type=jnp.float32,
                    precision=lax.Precision.HIGHEST)
        P = jnp.maximum(P, 0.0)
        xs = xs_ref[...]
        cols = [jnp.sum(P[:, o * IN:(o + 1) * IN] * xs, axis=1, keepdims=True)
                for o in range(HID)]
        cols.append(jnp.zeros((TE, MSGW - HID), jnp.float32))
        out_ref[...] = jnp.concatenate(cols, axis=1)

    msg = pl.pallas_call(
        edge_body,
        grid=(GE,),
        in_specs=[pl.BlockSpec((TE, KP), lambda i: (i, 0)),
                  pl.BlockSpec((TE, IN), lambda i: (i, 0)),
                  pl.BlockSpec((KP, HID * IN), lambda i: (0, 0))],
        out_specs=pl.BlockSpec((TE, MSGW), lambda i: (i, 0)),
        out_shape=jax.ShapeDtypeStruct((E, MSGW), jnp.float32),
    )(ea_aug, x_src, Wk)

    NPAD = ((N + NS * 8 - 1) // (NS * 8)) * NS * 8
    SUBS = CH // 16

    @functools.partial(
        pl.kernel,
        out_type=jax.ShapeDtypeStruct((NW, NPAD * HID), jnp.float32),
        mesh=mesh,
        scratch_types=[pltpu.VMEM((CH,), jnp.int32),
                       pltpu.VMEM((CH * MSGW,), jnp.float32),
                       pltpu.VMEM((16,), jnp.int32),
                       pltpu.VMEM((16 * MSGW,), jnp.float32),
                       pltpu.VMEM((NPAD * HID,), jnp.float32)],
        compiler_params=pltpu.CompilerParams(needs_layout_passes=False),
    )
    def scatter_k(msg_hbm, dst_hbm, zero_hbm, out_hbm,
                  idx_v, rows_v, idx_t, rows_t, agg_v):
        wid = lax.axis_index("c") * NS + lax.axis_index("s")
        pltpu.sync_copy(zero_hbm, agg_v)
        base = wid * EPW
        lanes = lax.iota(jnp.int32, 16)

        def chunk_body(i, c):
            off = base + i * CH
            pltpu.sync_copy(dst_hbm.at[pl.ds(off, CH)], idx_v)
            pltpu.sync_copy(msg_hbm.at[pl.ds(off * MSGW, CH * MSGW)], rows_v)

            def sub_body(j, c2):
                rowbase = j * 16
                dsub = idx_v[pl.ds(rowbase, 16)]
                vidx = (rowbase + lanes) * MSGW
                for o in range(HID):
                    vals = plsc.load_gather(rows_v, [vidx + o])
                    plsc.addupdate_scatter(agg_v, [dsub * HID + o], vals)
                return c2

            lax.fori_loop(0, SUBS, sub_body, 0)
            return c

        lax.fori_loop(0, NCHUNK, chunk_body, 0)
        if TAIL:
            off = base + NCHUNK * CH
            idx_t[...] = jnp.zeros((16,), jnp.int32)
            pltpu.sync_copy(dst_hbm.at[pl.ds(off, TAIL)],
                            idx_t.at[pl.ds(0, TAIL)])
            pltpu.sync_copy(msg_hbm.at[pl.ds(off * MSGW, TAIL * MSGW)],
                            rows_t.at[pl.ds(0, TAIL * MSGW)])
            dsub = idx_t[...]
            tmask = lanes < TAIL
            for o in range(HID):
                vals = plsc.load_gather(rows_t, [lanes * MSGW + o])
                plsc.addupdate_scatter(agg_v, [dsub * HID + o], vals,
                                       mask=tmask)
        pltpu.sync_copy(agg_v, out_hbm.at[wid])

    aggall = scatter_k(msg.reshape(E * MSGW), dst,
                       jnp.zeros((NPAD * HID,), jnp.float32))
    aggall = aggall.reshape(NW, NPAD, HID)

    TN = 1000
    GN = N // TN
    prm = jnp.zeros((8, 64), jnp.float32)
    prm = (prm.at[0, :HID].set(bconv).at[1, :HID].set(gamma)
              .at[2, :HID].set(beta).at[3, :OUT].set(blin)
              .at[4, :HDIM].set(bq1).at[5, :NA].set(bq2))

    def node_body(ag_ref, x_ref, root_ref, wlin_ref, wq1_ref,
                  wq2_ref, prm_ref, out_ref):
        p = prm_ref[...]
        agg = jnp.sum(ag_ref[...], axis=0)
        h = agg + jnp.dot(x_ref[...], root_ref[...],
                          preferred_element_type=jnp.float32,
                          precision=lax.Precision.HIGHEST) + p[0:1, :HID]
        mu = jnp.mean(h, axis=1, keepdims=True)
        var = jnp.mean((h - mu) ** 2, axis=1, keepdims=True)
        h = (h - mu) * lax.rsqrt(var + 1e-5) * p[1:2, :HID] + p[2:3, :HID]
        h = jnp.maximum(h, 0.0)
        h = jnp.dot(h, wlin_ref[...], preferred_element_type=jnp.float32,
                    precision=lax.Precision.HIGHEST) + p[3:4, :OUT]
        h = jnp.maximum(jnp.dot(h, wq1_ref[...],
                                preferred_element_type=jnp.float32,
                                precision=lax.Precision.HIGHEST)
                        + p[4:5, :HDIM], 0.0)
        out_ref[...] = jnp.dot(h, wq2_ref[...],
                               preferred_element_type=jnp.float32,
                               precision=lax.Precision.HIGHEST) + p[5:6, :NA]

    q = pl.pallas_call(
        node_body,
        grid=(GN,),
        in_specs=[pl.BlockSpec((NW, TN, HID), lambda i: (0, i, 0)),
                  pl.BlockSpec((TN, IN), lambda i: (i, 0)),
                  pl.BlockSpec((IN, HID), lambda i: (0, 0)),
                  pl.BlockSpec((HID, OUT), lambda i: (0, 0)),
                  pl.BlockSpec((OUT, HDIM), lambda i: (0, 0)),
                  pl.BlockSpec((HDIM, NA), lambda i: (0, 0)),
                  pl.BlockSpec((8, 64), lambda i: (0, 0))],
        out_specs=pl.BlockSpec((TN, NA), lambda i: (i, 0)),
        out_shape=jax.ShapeDtypeStruct((N, NA), jnp.float32),
    )(aggall, x, root, Wlin.T, Wq1.T, Wq2.T, prm)
    return q


# speedup vs baseline: 1.8174x; 1.4300x over previous
import functools

import jax
import jax.numpy as jnp
from jax import lax
from jax.experimental import pallas as pl
from jax.experimental.pallas import tpu as pltpu
from jax.experimental.pallas import tpu_sc as plsc

NC = 2
NS = 16
NW = NC * NS
CH = 128
MSGW = 16


def kernel(x, edge_index, edge_attr, W1, b1, root, bconv, gamma, beta,
           Wlin, blin, Wq1, bq1, Wq2, bq2):
    N, IN = x.shape
    E, EDIM = edge_attr.shape
    HID = root.shape[1]
    OUT = Wlin.shape[0]
    HDIM = Wq1.shape[0]
    NA = Wq2.shape[0]

    src = edge_index[0]
    dst = edge_index[1]

    KP = 24
    W1q = W1.reshape(IN, HID, EDIM).transpose(2, 1, 0).reshape(EDIM, HID * IN)
    b1q = b1.reshape(IN, HID).T.reshape(1, HID * IN)
    Wk = jnp.zeros((KP, HID * IN), jnp.float32)
    Wk = Wk.at[:EDIM].set(W1q).at[EDIM:EDIM + 1].set(b1q)
    ea_aug = jnp.concatenate(
        [edge_attr, jnp.ones((E, 1), jnp.float32),
         jnp.zeros((E, KP - EDIM - 1), jnp.float32)], axis=1)

    EPW = E // NW
    NCHUNK = EPW // CH
    TAIL = EPW - NCHUNK * CH

    mesh = plsc.VectorSubcoreMesh(core_axis_name="c", subcore_axis_name="s",
                                  num_cores=NC, num_subcores=NS)

    @functools.partial(
        pl.kernel,
        out_type=jax.ShapeDtypeStruct((E, IN), jnp.float32),
        mesh=mesh,
        scratch_types=[pltpu.VMEM((CH,), jnp.int32),
                       pltpu.VMEM((CH, IN), jnp.float32),
                       pltpu.VMEM((max(TAIL, 8),), jnp.int32),
                       pltpu.VMEM((max(TAIL, 8), IN), jnp.float32),
                       pltpu.SemaphoreType.DMA],
    )
    def gather_k(x_hbm, src_hbm, out_hbm, idx_v, rows_v, idx_t, rows_t, sem):
        wid = lax.axis_index("c") * NS + lax.axis_index("s")
        base = wid * EPW

        def body(i, c):
            off = base + i * CH
            pltpu.sync_copy(src_hbm.at[pl.ds(off, CH)], idx_v)
            pltpu.async_copy(x_hbm.at[idx_v], rows_v, sem).wait()
            pltpu.sync_copy(rows_v, out_hbm.at[pl.ds(off, CH)])
            return c

        lax.fori_loop(0, NCHUNK, body, 0)
        if TAIL:
            off = base + NCHUNK * CH
            pltpu.sync_copy(src_hbm.at[pl.ds(off, TAIL)], idx_t)
            pltpu.async_copy(x_hbm.at[idx_t], rows_t, sem).wait()
            pltpu.sync_copy(rows_t, out_hbm.at[pl.ds(off, TAIL)])

    x_src = gather_k(x, src)

    TE = 640
    GE = E // TE

    def edge_body(ea_ref, xs_ref, w_ref, out_ref):
        P = jnp.dot(ea_ref[...], w_ref[...],
                    preferred_element_type=jnp.float32)
        P = jnp.maximum(P, 0.0)
        xs = xs_ref[...]
        cols = [jnp.sum(P[:, o * IN:(o + 1) * IN] * xs, axis=1, keepdims=True)
                for o in range(HID)]
        cols.append(jnp.zeros((TE, MSGW - HID), jnp.float32))
        out_ref[...] = jnp.concatenate(cols, axis=1)

    msg = pl.pallas_call(
        edge_body,
        grid=(GE,),
        in_specs=[pl.BlockSpec((TE, KP), lambda i: (i, 0)),
                  pl.BlockSpec((TE, IN), lambda i: (i, 0)),
                  pl.BlockSpec((KP, HID * IN), lambda i: (0, 0))],
        out_specs=pl.BlockSpec((TE, MSGW), lambda i: (i, 0)),
        out_shape=jax.ShapeDtypeStruct((E, MSGW), jnp.float32),
    )(ea_aug, x_src, Wk)

    NPAD = ((N + NS * 8 - 1) // (NS * 8)) * NS * 8
    SUBS = CH // 16

    @functools.partial(
        pl.kernel,
        out_type=jax.ShapeDtypeStruct((NW, NPAD * HID), jnp.float32),
        mesh=mesh,
        scratch_types=[pltpu.VMEM((CH,), jnp.int32),
                       pltpu.VMEM((CH * MSGW,), jnp.float32),
                       pltpu.VMEM((16,), jnp.int32),
                       pltpu.VMEM((16 * MSGW,), jnp.float32),
                       pltpu.VMEM((NPAD * HID,), jnp.float32)],
        compiler_params=pltpu.CompilerParams(needs_layout_passes=False),
    )
    def scatter_k(msg_hbm, dst_hbm, zero_hbm, out_hbm,
                  idx_v, rows_v, idx_t, rows_t, agg_v):
        wid = lax.axis_index("c") * NS + lax.axis_index("s")
        pltpu.sync_copy(zero_hbm, agg_v)
        base = wid * EPW
        lanes = lax.iota(jnp.int32, 16)

        def chunk_body(i, c):
            off = base + i * CH
            pltpu.sync_copy(dst_hbm.at[pl.ds(off, CH)], idx_v)
            pltpu.sync_copy(msg_hbm.at[pl.ds(off * MSGW, CH * MSGW)], rows_v)

            def sub_body(j, c2):
                rowbase = j * 16
                dsub = idx_v[pl.ds(rowbase, 16)]
                vidx = (rowbase + lanes) * MSGW
                for o in range(HID):
                    vals = plsc.load_gather(rows_v, [vidx + o])
                    plsc.addupdate_scatter(agg_v, [dsub * HID + o], vals)
                return c2

            lax.fori_loop(0, SUBS, sub_body, 0)
            return c

        lax.fori_loop(0, NCHUNK, chunk_body, 0)
        if TAIL:
            off = base + NCHUNK * CH
            idx_t[...] = jnp.zeros((16,), jnp.int32)
            pltpu.sync_copy(dst_hbm.at[pl.ds(off, TAIL)],
                            idx_t.at[pl.ds(0, TAIL)])
            pltpu.sync_copy(msg_hbm.at[pl.ds(off * MSGW, TAIL * MSGW)],
                            rows_t.at[pl.ds(0, TAIL * MSGW)])
            dsub = idx_t[...]
            tmask = lanes < TAIL
            for o in range(HID):
                vals = plsc.load_gather(rows_t, [lanes * MSGW + o])
                plsc.addupdate_scatter(agg_v, [dsub * HID + o], vals,
                                       mask=tmask)
        pltpu.sync_copy(agg_v, out_hbm.at[wid])

    aggall = scatter_k(msg.reshape(E * MSGW), dst,
                       jnp.zeros((NPAD * HID,), jnp.float32))
    aggall = aggall.reshape(NW, NPAD, HID)

    TN = 1000
    GN = N // TN
    prm = jnp.zeros((8, 64), jnp.float32)
    prm = (prm.at[0, :HID].set(bconv).at[1, :HID].set(gamma)
              .at[2, :HID].set(beta).at[3, :OUT].set(blin)
              .at[4, :HDIM].set(bq1).at[5, :NA].set(bq2))

    def node_body(ag_ref, x_ref, root_ref, wlin_ref, wq1_ref,
                  wq2_ref, prm_ref, out_ref):
        p = prm_ref[...]
        agg = jnp.sum(ag_ref[...], axis=0)
        h = agg + jnp.dot(x_ref[...], root_ref[...],
                          preferred_element_type=jnp.float32,
                          precision=lax.Precision.HIGHEST) + p[0:1, :HID]
        mu = jnp.mean(h, axis=1, keepdims=True)
        var = jnp.mean((h - mu) ** 2, axis=1, keepdims=True)
        h = (h - mu) * lax.rsqrt(var + 1e-5) * p[1:2, :HID] + p[2:3, :HID]
        h = jnp.maximum(h, 0.0)
        h = jnp.dot(h, wlin_ref[...], preferred_element_type=jnp.float32,
                    precision=lax.Precision.HIGHEST) + p[3:4, :OUT]
        h = jnp.maximum(jnp.dot(h, wq1_ref[...],
                                preferred_element_type=jnp.float32,
                                precision=lax.Precision.HIGHEST)
                        + p[4:5, :HDIM], 0.0)
        out_ref[...] = jnp.dot(h, wq2_ref[...],
                               preferred_element_type=jnp.float32,
                               precision=lax.Precision.HIGHEST) + p[5:6, :NA]

    q = pl.pallas_call(
        node_body,
        grid=(GN,),
        in_specs=[pl.BlockSpec((NW, TN, HID), lambda i: (0, i, 0)),
                  pl.BlockSpec((TN, IN), lambda i: (i, 0)),
                  pl.BlockSpec((IN, HID), lambda i: (0, 0)),
                  pl.BlockSpec((HID, OUT), lambda i: (0, 0)),
                  pl.BlockSpec((OUT, HDIM), lambda i: (0, 0)),
                  pl.BlockSpec((HDIM, NA), lambda i: (0, 0)),
                  pl.BlockSpec((8, 64), lambda i: (0, 0))],
        out_specs=pl.BlockSpec((TN, NA), lambda i: (i, 0)),
        out_shape=jax.ShapeDtypeStruct((N, NA), jnp.float32),
    )(aggall, x, root, Wlin.T, Wq1.T, Wq2.T, prm)
    return q


# speedup vs baseline: 1.8656x; 1.0265x over previous
import functools

import jax
import jax.numpy as jnp
from jax import lax
from jax.experimental import pallas as pl
from jax.experimental.pallas import tpu as pltpu
from jax.experimental.pallas import tpu_sc as plsc

NC = 2
NS = 16
NW = NC * NS
CH = 128
MSGW = 16


def kernel(x, edge_index, edge_attr, W1, b1, root, bconv, gamma, beta,
           Wlin, blin, Wq1, bq1, Wq2, bq2):
    N, IN = x.shape
    E, EDIM = edge_attr.shape
    HID = root.shape[1]
    OUT = Wlin.shape[0]
    HDIM = Wq1.shape[0]
    NA = Wq2.shape[0]

    src = edge_index[0]
    dst = edge_index[1]

    KP = 24
    W1q = W1.reshape(IN, HID, EDIM).transpose(2, 1, 0).reshape(EDIM, HID * IN)
    b1q = b1.reshape(IN, HID).T.reshape(1, HID * IN)
    Wk = jnp.zeros((KP, HID * IN), jnp.float32)
    Wk = Wk.at[:EDIM].set(W1q).at[EDIM:EDIM + 1].set(b1q)
    ea_aug = jnp.concatenate(
        [edge_attr, jnp.ones((E, 1), jnp.float32),
         jnp.zeros((E, KP - EDIM - 1), jnp.float32)], axis=1)

    EPW = E // NW
    NCHUNK = EPW // CH
    TAIL = EPW - NCHUNK * CH

    mesh = plsc.VectorSubcoreMesh(core_axis_name="c", subcore_axis_name="s",
                                  num_cores=NC, num_subcores=NS)

    @functools.partial(
        pl.kernel,
        out_type=jax.ShapeDtypeStruct((E, IN), jnp.float32),
        mesh=mesh,
        scratch_types=[pltpu.VMEM((CH,), jnp.int32),
                       pltpu.VMEM((CH, IN), jnp.float32),
                       pltpu.VMEM((max(TAIL, 8),), jnp.int32),
                       pltpu.VMEM((max(TAIL, 8), IN), jnp.float32),
                       pltpu.SemaphoreType.DMA],
    )
    def gather_k(x_hbm, src_hbm, out_hbm, idx_v, rows_v, idx_t, rows_t, sem):
        wid = lax.axis_index("c") * NS + lax.axis_index("s")
        base = wid * EPW

        def body(i, c):
            off = base + i * CH
            pltpu.sync_copy(src_hbm.at[pl.ds(off, CH)], idx_v)
            pltpu.async_copy(x_hbm.at[idx_v], rows_v, sem).wait()
            pltpu.sync_copy(rows_v, out_hbm.at[pl.ds(off, CH)])
            return c

        lax.fori_loop(0, NCHUNK, body, 0)
        if TAIL:
            off = base + NCHUNK * CH
            pltpu.sync_copy(src_hbm.at[pl.ds(off, TAIL)], idx_t)
            pltpu.async_copy(x_hbm.at[idx_t], rows_t, sem).wait()
            pltpu.sync_copy(rows_t, out_hbm.at[pl.ds(off, TAIL)])

    x_src = gather_k(x, src)

    TE = 640
    GE = E // TE

    sel = (jnp.arange(HID * IN)[:, None] // IN ==
           jnp.arange(MSGW)[None, :]).astype(jnp.float32)

    def edge_body(ea_ref, xs_ref, w_ref, sel_ref, out_ref):
        P = jnp.dot(ea_ref[...], w_ref[...],
                    preferred_element_type=jnp.float32)
        P = jnp.maximum(P, 0.0)
        xs8 = jnp.concatenate([xs_ref[...]] * HID, axis=1)
        out_ref[...] = jnp.dot(P * xs8, sel_ref[...],
                               preferred_element_type=jnp.float32)

    msg = pl.pallas_call(
        edge_body,
        grid=(GE,),
        in_specs=[pl.BlockSpec((TE, KP), lambda i: (i, 0)),
                  pl.BlockSpec((TE, IN), lambda i: (i, 0)),
                  pl.BlockSpec((KP, HID * IN), lambda i: (0, 0)),
                  pl.BlockSpec((HID * IN, MSGW), lambda i: (0, 0))],
        out_specs=pl.BlockSpec((TE, MSGW), lambda i: (i, 0)),
        out_shape=jax.ShapeDtypeStruct((E, MSGW), jnp.float32),
    )(ea_aug, x_src, Wk, sel)

    NPAD = ((N + NS * 8 - 1) // (NS * 8)) * NS * 8
    SUBS = CH // 16

    @functools.partial(
        pl.kernel,
        out_type=jax.ShapeDtypeStruct((NW, NPAD * HID), jnp.float32),
        mesh=mesh,
        scratch_types=[pltpu.VMEM((CH,), jnp.int32),
                       pltpu.VMEM((CH * MSGW,), jnp.float32),
                       pltpu.VMEM((16,), jnp.int32),
                       pltpu.VMEM((16 * MSGW,), jnp.float32),
                       pltpu.VMEM((NPAD * HID,), jnp.float32)],
        compiler_params=pltpu.CompilerParams(needs_layout_passes=False),
    )
    def scatter_k(msg_hbm, dst_hbm, zero_hbm, out_hbm,
                  idx_v, rows_v, idx_t, rows_t, agg_v):
        wid = lax.axis_index("c") * NS + lax.axis_index("s")
        pltpu.sync_copy(zero_hbm, agg_v)
        base = wid * EPW
        lanes = lax.iota(jnp.int32, 16)

        def chunk_body(i, c):
            off = base + i * CH
            pltpu.sync_copy(dst_hbm.at[pl.ds(off, CH)], idx_v)
            pltpu.sync_copy(msg_hbm.at[pl.ds(off * MSGW, CH * MSGW)], rows_v)

            def sub_body(j, c2):
                rowbase = j * 16
                dsub = idx_v[pl.ds(rowbase, 16)]
                vidx = (rowbase + lanes) * MSGW
                for o in range(HID):
                    vals = plsc.load_gather(rows_v, [vidx + o])
                    plsc.addupdate_scatter(agg_v, [dsub * HID + o], vals)
                return c2

            lax.fori_loop(0, SUBS, sub_body, 0)
            return c

        lax.fori_loop(0, NCHUNK, chunk_body, 0)
        if TAIL:
            off = base + NCHUNK * CH
            idx_t[...] = jnp.zeros((16,), jnp.int32)
            pltpu.sync_copy(dst_hbm.at[pl.ds(off, TAIL)],
                            idx_t.at[pl.ds(0, TAIL)])
            pltpu.sync_copy(msg_hbm.at[pl.ds(off * MSGW, TAIL * MSGW)],
                            rows_t.at[pl.ds(0, TAIL * MSGW)])
            dsub = idx_t[...]
            tmask = lanes < TAIL
            for o in range(HID):
                vals = plsc.load_gather(rows_t, [lanes * MSGW + o])
                plsc.addupdate_scatter(agg_v, [dsub * HID + o], vals,
                                       mask=tmask)
        pltpu.sync_copy(agg_v, out_hbm.at[wid])

    aggall = scatter_k(msg.reshape(E * MSGW), dst,
                       jnp.zeros((NPAD * HID,), jnp.float32))
    aggall = aggall.reshape(NW, NPAD, HID)

    TN = 1000
    GN = N // TN
    prm = jnp.zeros((8, 64), jnp.float32)
    prm = (prm.at[0, :HID].set(bconv).at[1, :HID].set(gamma)
              .at[2, :HID].set(beta).at[3, :OUT].set(blin)
              .at[4, :HDIM].set(bq1).at[5, :NA].set(bq2))

    def node_body(ag_ref, x_ref, root_ref, wlin_ref, wq1_ref,
                  wq2_ref, prm_ref, out_ref):
        p = prm_ref[...]
        agg = jnp.sum(ag_ref[...], axis=0)
        h = agg + jnp.dot(x_ref[...], root_ref[...],
                          preferred_element_type=jnp.float32) + p[0:1, :HID]
        mu = jnp.mean(h, axis=1, keepdims=True)
        var = jnp.mean((h - mu) ** 2, axis=1, keepdims=True)
        h = (h - mu) * lax.rsqrt(var + 1e-5) * p[1:2, :HID] + p[2:3, :HID]
        h = jnp.maximum(h, 0.0)
        h = jnp.dot(h, wlin_ref[...],
                    preferred_element_type=jnp.float32) + p[3:4, :OUT]
        h = jnp.maximum(jnp.dot(h, wq1_ref[...],
                                preferred_element_type=jnp.float32)
                        + p[4:5, :HDIM], 0.0)
        out_ref[...] = jnp.dot(h, wq2_ref[...],
                               preferred_element_type=jnp.float32) + p[5:6, :NA]

    q = pl.pallas_call(
        node_body,
        grid=(GN,),
        in_specs=[pl.BlockSpec((NW, TN, HID), lambda i: (0, i, 0)),
                  pl.BlockSpec((TN, IN), lambda i: (i, 0)),
                  pl.BlockSpec((IN, HID), lambda i: (0, 0)),
                  pl.BlockSpec((HID, OUT), lambda i: (0, 0)),
                  pl.BlockSpec((OUT, HDIM), lambda i: (0, 0)),
                  pl.BlockSpec((HDIM, NA), lambda i: (0, 0)),
                  pl.BlockSpec((8, 64), lambda i: (0, 0))],
        out_specs=pl.BlockSpec((TN, NA), lambda i: (i, 0)),
        out_shape=jax.ShapeDtypeStruct((N, NA), jnp.float32),
    )(aggall, x, root, Wlin.T, Wq1.T, Wq2.T, prm)
    return q


# speedup vs baseline: 1.8774x; 1.0063x over previous
import functools

import jax
import jax.numpy as jnp
from jax import lax
from jax.experimental import pallas as pl
from jax.experimental.pallas import tpu as pltpu
from jax.experimental.pallas import tpu_sc as plsc

NC = 2
NS = 16
NW = NC * NS
CH = 128
MSGW = 16


def kernel(x, edge_index, edge_attr, W1, b1, root, bconv, gamma, beta,
           Wlin, blin, Wq1, bq1, Wq2, bq2):
    N, IN = x.shape
    E, EDIM = edge_attr.shape
    HID = root.shape[1]
    OUT = Wlin.shape[0]
    HDIM = Wq1.shape[0]
    NA = Wq2.shape[0]

    src = edge_index[0]
    dst = edge_index[1]

    W1q = W1.reshape(IN, HID, EDIM).transpose(2, 1, 0).reshape(EDIM, HID * IN)
    b1q = jnp.broadcast_to(b1.reshape(IN, HID).T.reshape(1, HID * IN),
                           (8, HID * IN))

    EPW = E // NW
    NCHUNK = EPW // CH
    TAIL = EPW - NCHUNK * CH

    mesh = plsc.VectorSubcoreMesh(core_axis_name="c", subcore_axis_name="s",
                                  num_cores=NC, num_subcores=NS)

    @functools.partial(
        pl.kernel,
        out_type=jax.ShapeDtypeStruct((E, IN), jnp.float32),
        mesh=mesh,
        scratch_types=[pltpu.VMEM((CH,), jnp.int32),
                       pltpu.VMEM((CH, IN), jnp.float32),
                       pltpu.VMEM((max(TAIL, 8),), jnp.int32),
                       pltpu.VMEM((max(TAIL, 8), IN), jnp.float32),
                       pltpu.SemaphoreType.DMA],
    )
    def gather_k(x_hbm, src_hbm, out_hbm, idx_v, rows_v, idx_t, rows_t, sem):
        wid = lax.axis_index("c") * NS + lax.axis_index("s")
        base = wid * EPW

        def body(i, c):
            off = base + i * CH
            pltpu.sync_copy(src_hbm.at[pl.ds(off, CH)], idx_v)
            pltpu.async_copy(x_hbm.at[idx_v], rows_v, sem).wait()
            pltpu.sync_copy(rows_v, out_hbm.at[pl.ds(off, CH)])
            return c

        lax.fori_loop(0, NCHUNK, body, 0)
        if TAIL:
            off = base + NCHUNK * CH
            pltpu.sync_copy(src_hbm.at[pl.ds(off, TAIL)], idx_t)
            pltpu.async_copy(x_hbm.at[idx_t], rows_t, sem).wait()
            pltpu.sync_copy(rows_t, out_hbm.at[pl.ds(off, TAIL)])

    x_src = gather_k(x, src)

    TE = 640
    GE = E // TE

    sel = (jnp.arange(HID * IN)[:, None] // IN ==
           jnp.arange(MSGW)[None, :]).astype(jnp.float32)

    def edge_body(ea_ref, xs_ref, w_ref, b_ref, sel_ref, out_ref):
        P = jnp.dot(ea_ref[...], w_ref[...],
                    preferred_element_type=jnp.float32) + b_ref[0:1, :]
        P = jnp.maximum(P, 0.0)
        xs8 = jnp.concatenate([xs_ref[...]] * HID, axis=1)
        out_ref[...] = jnp.dot(P * xs8, sel_ref[...],
                               preferred_element_type=jnp.float32)

    msg = pl.pallas_call(
        edge_body,
        grid=(GE,),
        in_specs=[pl.BlockSpec((TE, EDIM), lambda i: (i, 0)),
                  pl.BlockSpec((TE, IN), lambda i: (i, 0)),
                  pl.BlockSpec((EDIM, HID * IN), lambda i: (0, 0)),
                  pl.BlockSpec((8, HID * IN), lambda i: (0, 0)),
                  pl.BlockSpec((HID * IN, MSGW), lambda i: (0, 0))],
        out_specs=pl.BlockSpec((TE, MSGW), lambda i: (i, 0)),
        out_shape=jax.ShapeDtypeStruct((E, MSGW), jnp.float32),
    )(edge_attr, x_src, W1q, b1q, sel)

    NPAD = ((N + NS * 8 - 1) // (NS * 8)) * NS * 8
    SUBS = CH // 16

    @functools.partial(
        pl.kernel,
        out_type=jax.ShapeDtypeStruct((NW, NPAD * HID), jnp.float32),
        mesh=mesh,
        scratch_types=[pltpu.VMEM((CH,), jnp.int32),
                       pltpu.VMEM((CH * MSGW,), jnp.float32),
                       pltpu.VMEM((16,), jnp.int32),
                       pltpu.VMEM((16 * MSGW,), jnp.float32),
                       pltpu.VMEM((NPAD * HID,), jnp.float32)],
        compiler_params=pltpu.CompilerParams(needs_layout_passes=False),
    )
    def scatter_k(msg_hbm, dst_hbm, zero_hbm, out_hbm,
                  idx_v, rows_v, idx_t, rows_t, agg_v):
        wid = lax.axis_index("c") * NS + lax.axis_index("s")
        pltpu.sync_copy(zero_hbm, agg_v)
        base = wid * EPW
        lanes = lax.iota(jnp.int32, 16)

        def chunk_body(i, c):
            off = base + i * CH
            pltpu.sync_copy(dst_hbm.at[pl.ds(off, CH)], idx_v)
            pltpu.sync_copy(msg_hbm.at[pl.ds(off * MSGW, CH * MSGW)], rows_v)

            def sub_body(j, c2):
                rowbase = j * 16
                dsub = idx_v[pl.ds(rowbase, 16)]
                vidx = (rowbase + lanes) * MSGW
                for o in range(HID):
                    vals = plsc.load_gather(rows_v, [vidx + o])
                    plsc.addupdate_scatter(agg_v, [dsub * HID + o], vals)
                return c2

            lax.fori_loop(0, SUBS, sub_body, 0)
            return c

        lax.fori_loop(0, NCHUNK, chunk_body, 0)
        if TAIL:
            off = base + NCHUNK * CH
            idx_t[...] = jnp.zeros((16,), jnp.int32)
            pltpu.sync_copy(dst_hbm.at[pl.ds(off, TAIL)],
                            idx_t.at[pl.ds(0, TAIL)])
            pltpu.sync_copy(msg_hbm.at[pl.ds(off * MSGW, TAIL * MSGW)],
                            rows_t.at[pl.ds(0, TAIL * MSGW)])
            dsub = idx_t[...]
            tmask = lanes < TAIL
            for o in range(HID):
                vals = plsc.load_gather(rows_t, [lanes * MSGW + o])
                plsc.addupdate_scatter(agg_v, [dsub * HID + o], vals,
                                       mask=tmask)
        pltpu.sync_copy(agg_v, out_hbm.at[wid])

    aggall = scatter_k(msg.reshape(E * MSGW), dst,
                       jnp.zeros((NPAD * HID,), jnp.float32))
    aggall = aggall.reshape(NW, NPAD, HID)

    TN = 1000
    GN = N // TN
    prm = jnp.zeros((8, 64), jnp.float32)
    prm = (prm.at[0, :HID].set(bconv).at[1, :HID].set(gamma)
              .at[2, :HID].set(beta).at[3, :OUT].set(blin)
              .at[4, :HDIM].set(bq1).at[5, :NA].set(bq2))

    def node_body(ag_ref, x_ref, root_ref, wlin_ref, wq1_ref,
                  wq2_ref, prm_ref, out_ref):
        p = prm_ref[...]
        agg = jnp.sum(ag_ref[...], axis=0)
        h = agg + jnp.dot(x_ref[...], root_ref[...],
                          preferred_element_type=jnp.float32) + p[0:1, :HID]
        mu = jnp.mean(h, axis=1, keepdims=True)
        var = jnp.mean((h - mu) ** 2, axis=1, keepdims=True)
        h = (h - mu) * lax.rsqrt(var + 1e-5) * p[1:2, :HID] + p[2:3, :HID]
        h = jnp.maximum(h, 0.0)
        h = jnp.dot(h, wlin_ref[...],
                    preferred_element_type=jnp.float32) + p[3:4, :OUT]
        h = jnp.maximum(jnp.dot(h, wq1_ref[...],
                                preferred_element_type=jnp.float32)
                        + p[4:5, :HDIM], 0.0)
        out_ref[...] = jnp.dot(h, wq2_ref[...],
                               preferred_element_type=jnp.float32) + p[5:6, :NA]

    q = pl.pallas_call(
        node_body,
        grid=(GN,),
        in_specs=[pl.BlockSpec((NW, TN, HID), lambda i: (0, i, 0)),
                  pl.BlockSpec((TN, IN), lambda i: (i, 0)),
                  pl.BlockSpec((IN, HID), lambda i: (0, 0)),
                  pl.BlockSpec((HID, OUT), lambda i: (0, 0)),
                  pl.BlockSpec((OUT, HDIM), lambda i: (0, 0)),
                  pl.BlockSpec((HDIM, NA), lambda i: (0, 0)),
                  pl.BlockSpec((8, 64), lambda i: (0, 0))],
        out_specs=pl.BlockSpec((TN, NA), lambda i: (i, 0)),
        out_shape=jax.ShapeDtypeStruct((N, NA), jnp.float32),
    )(aggall, x, root, Wlin.T, Wq1.T, Wq2.T, prm)
    return q


# speedup vs baseline: 1.9238x; 1.0247x over previous
import functools

import jax
import jax.numpy as jnp
from jax import lax
from jax.experimental import pallas as pl
from jax.experimental.pallas import tpu as pltpu
from jax.experimental.pallas import tpu_sc as plsc

NC = 2
NS = 16
NW = NC * NS
CH = 128
MSGW = 16


def kernel(x, edge_index, edge_attr, W1, b1, root, bconv, gamma, beta,
           Wlin, blin, Wq1, bq1, Wq2, bq2):
    N, IN = x.shape
    E, EDIM = edge_attr.shape
    HID = root.shape[1]
    OUT = Wlin.shape[0]
    HDIM = Wq1.shape[0]
    NA = Wq2.shape[0]

    src = edge_index[0]
    dst = edge_index[1]

    W1q = W1.reshape(IN, HID, EDIM).transpose(2, 1, 0).reshape(EDIM, HID * IN)
    b1q = jnp.broadcast_to(b1.reshape(IN, HID).T.reshape(1, HID * IN),
                           (8, HID * IN))

    EPW = E // NW
    NCHUNK = EPW // CH
    TAIL = EPW - NCHUNK * CH

    mesh = plsc.VectorSubcoreMesh(core_axis_name="c", subcore_axis_name="s",
                                  num_cores=NC, num_subcores=NS)

    @functools.partial(
        pl.kernel,
        out_type=jax.ShapeDtypeStruct((E, IN), jnp.float32),
        mesh=mesh,
        scratch_types=[pltpu.VMEM((CH,), jnp.int32),
                       pltpu.VMEM((CH,), jnp.int32),
                       pltpu.VMEM((CH, IN), jnp.float32),
                       pltpu.VMEM((CH, IN), jnp.float32),
                       pltpu.VMEM((max(TAIL, 8),), jnp.int32),
                       pltpu.VMEM((max(TAIL, 8), IN), jnp.float32),
                       pltpu.SemaphoreType.DMA,
                       pltpu.SemaphoreType.DMA,
                       pltpu.SemaphoreType.DMA,
                       pltpu.SemaphoreType.DMA],
    )
    def gather_k(x_hbm, src_hbm, out_hbm, idx_a, idx_b, rows_a, rows_b,
                 idx_t, rows_t, sem_ga, sem_gb, sem_wa, sem_wb):
        wid = lax.axis_index("c") * NS + lax.axis_index("s")
        base = wid * EPW
        NPAIR = NCHUNK // 2

        def pair_body(i, c):
            off_a = base + (2 * i) * CH
            off_b = off_a + CH
            pltpu.sync_copy(src_hbm.at[pl.ds(off_a, CH)], idx_a)
            ga = pltpu.async_copy(x_hbm.at[idx_a], rows_a, sem_ga)
            pltpu.sync_copy(src_hbm.at[pl.ds(off_b, CH)], idx_b)
            ga.wait()
            gb = pltpu.async_copy(x_hbm.at[idx_b], rows_b, sem_gb)
            wa = pltpu.async_copy(rows_a, out_hbm.at[pl.ds(off_a, CH)],
                                  sem_wa)
            gb.wait()
            wb = pltpu.async_copy(rows_b, out_hbm.at[pl.ds(off_b, CH)],
                                  sem_wb)
            wa.wait()
            wb.wait()
            return c

        lax.fori_loop(0, NPAIR, pair_body, 0)
        if NCHUNK % 2:
            off = base + (NCHUNK - 1) * CH
            pltpu.sync_copy(src_hbm.at[pl.ds(off, CH)], idx_a)
            pltpu.async_copy(x_hbm.at[idx_a], rows_a, sem_ga).wait()
            pltpu.sync_copy(rows_a, out_hbm.at[pl.ds(off, CH)])
        if TAIL:
            off = base + NCHUNK * CH
            pltpu.sync_copy(src_hbm.at[pl.ds(off, TAIL)], idx_t)
            pltpu.async_copy(x_hbm.at[idx_t], rows_t, sem_ga).wait()
            pltpu.sync_copy(rows_t, out_hbm.at[pl.ds(off, TAIL)])

    x_src = gather_k(x, src)

    TE = 640
    GE = E // TE

    sel = (jnp.arange(HID * IN)[:, None] // IN ==
           jnp.arange(MSGW)[None, :]).astype(jnp.float32)

    def edge_body(ea_ref, xs_ref, w_ref, b_ref, sel_ref, out_ref):
        P = jnp.dot(ea_ref[...], w_ref[...],
                    preferred_element_type=jnp.float32) + b_ref[0:1, :]
        P = jnp.maximum(P, 0.0)
        xs8 = jnp.concatenate([xs_ref[...]] * HID, axis=1)
        out_ref[...] = jnp.dot(P * xs8, sel_ref[...],
                               preferred_element_type=jnp.float32)

    msg = pl.pallas_call(
        edge_body,
        grid=(GE,),
        in_specs=[pl.BlockSpec((TE, EDIM), lambda i: (i, 0)),
                  pl.BlockSpec((TE, IN), lambda i: (i, 0)),
                  pl.BlockSpec((EDIM, HID * IN), lambda i: (0, 0)),
                  pl.BlockSpec((8, HID * IN), lambda i: (0, 0)),
                  pl.BlockSpec((HID * IN, MSGW), lambda i: (0, 0))],
        out_specs=pl.BlockSpec((TE, MSGW), lambda i: (i, 0)),
        out_shape=jax.ShapeDtypeStruct((E, MSGW), jnp.float32),
    )(edge_attr, x_src, W1q, b1q, sel)

    NPAD = ((N + NS * 8 - 1) // (NS * 8)) * NS * 8
    SUBS = CH // 16

    @functools.partial(
        pl.kernel,
        out_type=jax.ShapeDtypeStruct((NW, NPAD * HID), jnp.float32),
        mesh=mesh,
        scratch_types=[pltpu.VMEM((CH,), jnp.int32),
                       pltpu.VMEM((CH * MSGW,), jnp.float32),
                       pltpu.VMEM((16,), jnp.int32),
                       pltpu.VMEM((16 * MSGW,), jnp.float32),
                       pltpu.VMEM((NPAD * HID,), jnp.float32)],
        compiler_params=pltpu.CompilerParams(needs_layout_passes=False),
    )
    def scatter_k(msg_hbm, dst_hbm, zero_hbm, out_hbm,
                  idx_v, rows_v, idx_t, rows_t, agg_v):
        wid = lax.axis_index("c") * NS + lax.axis_index("s")
        pltpu.sync_copy(zero_hbm, agg_v)
        base = wid * EPW
        lanes = lax.iota(jnp.int32, 16)

        def chunk_body(i, c):
            off = base + i * CH
            pltpu.sync_copy(dst_hbm.at[pl.ds(off, CH)], idx_v)
            pltpu.sync_copy(msg_hbm.at[pl.ds(off * MSGW, CH * MSGW)], rows_v)

            def sub_body(j, c2):
                rowbase = j * 16
                dsub = idx_v[pl.ds(rowbase, 16)]
                vidx = (rowbase + lanes) * MSGW
                for o in range(HID):
                    vals = plsc.load_gather(rows_v, [vidx + o])
                    plsc.addupdate_scatter(agg_v, [dsub * HID + o], vals)
                return c2

            lax.fori_loop(0, SUBS, sub_body, 0)
            return c

        lax.fori_loop(0, NCHUNK, chunk_body, 0)
        if TAIL:
            off = base + NCHUNK * CH
            idx_t[...] = jnp.zeros((16,), jnp.int32)
            pltpu.sync_copy(dst_hbm.at[pl.ds(off, TAIL)],
                            idx_t.at[pl.ds(0, TAIL)])
            pltpu.sync_copy(msg_hbm.at[pl.ds(off * MSGW, TAIL * MSGW)],
                            rows_t.at[pl.ds(0, TAIL * MSGW)])
            dsub = idx_t[...]
            tmask = lanes < TAIL
            for o in range(HID):
                vals = plsc.load_gather(rows_t, [lanes * MSGW + o])
                plsc.addupdate_scatter(agg_v, [dsub * HID + o], vals,
                                       mask=tmask)
        pltpu.sync_copy(agg_v, out_hbm.at[wid])

    aggall = scatter_k(msg.reshape(E * MSGW), dst,
                       jnp.zeros((NPAD * HID,), jnp.float32))
    aggall = aggall.reshape(NW, NPAD, HID)

    TN = 1000
    GN = N // TN
    prm = jnp.zeros((8, 64), jnp.float32)
    prm = (prm.at[0, :HID].set(bconv).at[1, :HID].set(gamma)
              .at[2, :HID].set(beta).at[3, :OUT].set(blin)
              .at[4, :HDIM].set(bq1).at[5, :NA].set(bq2))

    def node_body(ag_ref, x_ref, root_ref, wlin_ref, wq1_ref,
                  wq2_ref, prm_ref, out_ref):
        p = prm_ref[...]
        agg = jnp.sum(ag_ref[...], axis=0)
        h = agg + jnp.dot(x_ref[...], root_ref[...],
                          preferred_element_type=jnp.float32) + p[0:1, :HID]
        mu = jnp.mean(h, axis=1, keepdims=True)
        var = jnp.mean((h - mu) ** 2, axis=1, keepdims=True)
        h = (h - mu) * lax.rsqrt(var + 1e-5) * p[1:2, :HID] + p[2:3, :HID]
        h = jnp.maximum(h, 0.0)
        h = jnp.dot(h, wlin_ref[...],
                    preferred_element_type=jnp.float32) + p[3:4, :OUT]
        h = jnp.maximum(jnp.dot(h, wq1_ref[...],
                                preferred_element_type=jnp.float32)
                        + p[4:5, :HDIM], 0.0)
        out_ref[...] = jnp.dot(h, wq2_ref[...],
                               preferred_element_type=jnp.float32) + p[5:6, :NA]

    q = pl.pallas_call(
        node_body,
        grid=(GN,),
        in_specs=[pl.BlockSpec((NW, TN, HID), lambda i: (0, i, 0)),
                  pl.BlockSpec((TN, IN), lambda i: (i, 0)),
                  pl.BlockSpec((IN, HID), lambda i: (0, 0)),
                  pl.BlockSpec((HID, OUT), lambda i: (0, 0)),
                  pl.BlockSpec((OUT, HDIM), lambda i: (0, 0)),
                  pl.BlockSpec((HDIM, NA), lambda i: (0, 0)),
                  pl.BlockSpec((8, 64), lambda i: (0, 0))],
        out_specs=pl.BlockSpec((TN, NA), lambda i: (i, 0)),
        out_shape=jax.ShapeDtypeStruct((N, NA), jnp.float32),
    )(aggall, x, root, Wlin.T, Wq1.T, Wq2.T, prm)
    return q


# speedup vs baseline: 2.1344x; 1.1095x over previous
import functools

import jax
import jax.numpy as jnp
from jax import lax
from jax.experimental import pallas as pl
from jax.experimental.pallas import tpu as pltpu
from jax.experimental.pallas import tpu_sc as plsc

NC = 2
NS = 16
NW = NC * NS
CH = 128
MSGW = 16


def kernel(x, edge_index, edge_attr, W1, b1, root, bconv, gamma, beta,
           Wlin, blin, Wq1, bq1, Wq2, bq2):
    N, IN = x.shape
    E, EDIM = edge_attr.shape
    HID = root.shape[1]
    OUT = Wlin.shape[0]
    HDIM = Wq1.shape[0]
    NA = Wq2.shape[0]

    src = edge_index[0]
    dst = edge_index[1]

    W1q = W1.reshape(IN, HID, EDIM).transpose(2, 1, 0).reshape(EDIM, HID * IN)
    b1q = jnp.broadcast_to(b1.reshape(IN, HID).T.reshape(1, HID * IN),
                           (8, HID * IN))

    EPW = E // NW
    NCHUNK = EPW // CH
    TAIL = EPW - NCHUNK * CH

    mesh = plsc.VectorSubcoreMesh(core_axis_name="c", subcore_axis_name="s",
                                  num_cores=NC, num_subcores=NS)

    @functools.partial(
        pl.kernel,
        out_type=jax.ShapeDtypeStruct((E, IN), jnp.float32),
        mesh=mesh,
        scratch_types=[pltpu.VMEM((CH,), jnp.int32),
                       pltpu.VMEM((CH,), jnp.int32),
                       pltpu.VMEM((CH, IN), jnp.float32),
                       pltpu.VMEM((CH, IN), jnp.float32),
                       pltpu.VMEM((max(TAIL, 8),), jnp.int32),
                       pltpu.VMEM((max(TAIL, 8), IN), jnp.float32),
                       pltpu.SemaphoreType.DMA,
                       pltpu.SemaphoreType.DMA,
                       pltpu.SemaphoreType.DMA,
                       pltpu.SemaphoreType.DMA],
    )
    def gather_k(x_hbm, src_hbm, out_hbm, idx_a, idx_b, rows_a, rows_b,
                 idx_t, rows_t, sem_ga, sem_gb, sem_wa, sem_wb):
        wid = lax.axis_index("c") * NS + lax.axis_index("s")
        base = wid * EPW
        NPAIR = NCHUNK // 2

        def pair_body(i, c):
            off_a = base + (2 * i) * CH
            off_b = off_a + CH
            pltpu.sync_copy(src_hbm.at[pl.ds(off_a, CH)], idx_a)
            ga = pltpu.async_copy(x_hbm.at[idx_a], rows_a, sem_ga)
            pltpu.sync_copy(src_hbm.at[pl.ds(off_b, CH)], idx_b)
            ga.wait()
            gb = pltpu.async_copy(x_hbm.at[idx_b], rows_b, sem_gb)
            wa = pltpu.async_copy(rows_a, out_hbm.at[pl.ds(off_a, CH)],
                                  sem_wa)
            gb.wait()
            wb = pltpu.async_copy(rows_b, out_hbm.at[pl.ds(off_b, CH)],
                                  sem_wb)
            wa.wait()
            wb.wait()
            return c

        lax.fori_loop(0, NPAIR, pair_body, 0)
        if NCHUNK % 2:
            off = base + (NCHUNK - 1) * CH
            pltpu.sync_copy(src_hbm.at[pl.ds(off, CH)], idx_a)
            pltpu.async_copy(x_hbm.at[idx_a], rows_a, sem_ga).wait()
            pltpu.sync_copy(rows_a, out_hbm.at[pl.ds(off, CH)])
        if TAIL:
            off = base + NCHUNK * CH
            pltpu.sync_copy(src_hbm.at[pl.ds(off, TAIL)], idx_t)
            pltpu.async_copy(x_hbm.at[idx_t], rows_t, sem_ga).wait()
            pltpu.sync_copy(rows_t, out_hbm.at[pl.ds(off, TAIL)])

    x_src = gather_k(x, src)

    TE = 1280
    GE = E // TE

    sel = (jnp.arange(HID * IN)[:, None] // IN ==
           jnp.arange(MSGW)[None, :]).astype(jnp.float32)

    def edge_body(ea_ref, xs_ref, w_ref, b_ref, sel_ref, out_ref):
        P = jnp.dot(ea_ref[...], w_ref[...],
                    preferred_element_type=jnp.float32) + b_ref[0:1, :]
        P = jnp.maximum(P, 0.0)
        xs8 = jnp.concatenate([xs_ref[...]] * HID, axis=1)
        out_ref[...] = jnp.dot(P * xs8, sel_ref[...],
                               preferred_element_type=jnp.float32)

    msg = pl.pallas_call(
        edge_body,
        grid=(GE,),
        in_specs=[pl.BlockSpec((TE, EDIM), lambda i: (i, 0)),
                  pl.BlockSpec((TE, IN), lambda i: (i, 0)),
                  pl.BlockSpec((EDIM, HID * IN), lambda i: (0, 0)),
                  pl.BlockSpec((8, HID * IN), lambda i: (0, 0)),
                  pl.BlockSpec((HID * IN, MSGW), lambda i: (0, 0))],
        out_specs=pl.BlockSpec((TE, MSGW), lambda i: (i, 0)),
        out_shape=jax.ShapeDtypeStruct((E, MSGW), jnp.float32),
    )(edge_attr, x_src, W1q, b1q, sel)

    NPAD = ((N + NS * 8 - 1) // (NS * 8)) * NS * 8
    SUBS = CH // 16

    @functools.partial(
        pl.kernel,
        out_type=jax.ShapeDtypeStruct((NW, NPAD * HID), jnp.float32),
        mesh=mesh,
        scratch_types=[pltpu.VMEM((CH,), jnp.int32),
                       pltpu.VMEM((CH * MSGW,), jnp.float32),
                       pltpu.VMEM((16,), jnp.int32),
                       pltpu.VMEM((16 * MSGW,), jnp.float32),
                       pltpu.VMEM((NPAD * HID,), jnp.float32)],
        compiler_params=pltpu.CompilerParams(needs_layout_passes=False),
    )
    def scatter_k(msg_hbm, dst_hbm, zero_hbm, out_hbm,
                  idx_v, rows_v, idx_t, rows_t, agg_v):
        wid = lax.axis_index("c") * NS + lax.axis_index("s")
        pltpu.sync_copy(zero_hbm, agg_v)
        base = wid * EPW
        lanes = lax.iota(jnp.int32, 16)

        def chunk_body(i, c):
            off = base + i * CH
            pltpu.sync_copy(dst_hbm.at[pl.ds(off, CH)], idx_v)
            pltpu.sync_copy(msg_hbm.at[pl.ds(off * MSGW, CH * MSGW)], rows_v)

            def sub_body(j, c2):
                rowbase = j * 16
                dsub = idx_v[pl.ds(rowbase, 16)]
                vidx = (rowbase + lanes) * MSGW
                for o in range(HID):
                    vals = plsc.load_gather(rows_v, [vidx + o])
                    plsc.addupdate_scatter(agg_v, [dsub * HID + o], vals)
                return c2

            lax.fori_loop(0, SUBS, sub_body, 0)
            return c

        lax.fori_loop(0, NCHUNK, chunk_body, 0)
        if TAIL:
            off = base + NCHUNK * CH
            idx_t[...] = jnp.zeros((16,), jnp.int32)
            pltpu.sync_copy(dst_hbm.at[pl.ds(off, TAIL)],
                            idx_t.at[pl.ds(0, TAIL)])
            pltpu.sync_copy(msg_hbm.at[pl.ds(off * MSGW, TAIL * MSGW)],
                            rows_t.at[pl.ds(0, TAIL * MSGW)])
            dsub = idx_t[...]
            tmask = lanes < TAIL
            for o in range(HID):
                vals = plsc.load_gather(rows_t, [lanes * MSGW + o])
                plsc.addupdate_scatter(agg_v, [dsub * HID + o], vals,
                                       mask=tmask)
        pltpu.sync_copy(agg_v, out_hbm.at[wid])

    aggall = scatter_k(msg.reshape(E * MSGW), dst,
                       jnp.zeros((NPAD * HID,), jnp.float32))
    aggall = aggall.reshape(NW, NPAD, HID)

    TN = 1000
    GN = N // TN
    prm = jnp.zeros((8, 64), jnp.float32)
    prm = (prm.at[0, :HID].set(bconv).at[1, :HID].set(gamma)
              .at[2, :HID].set(beta).at[3, :OUT].set(blin)
              .at[4, :HDIM].set(bq1).at[5, :NA].set(bq2))

    def node_body(ag_ref, x_ref, root_ref, wlin_ref, wq1_ref,
                  wq2_ref, prm_ref, out_ref):
        p = prm_ref[...]
        agg = jnp.sum(ag_ref[...], axis=0)
        h = agg + jnp.dot(x_ref[...], root_ref[...],
                          preferred_element_type=jnp.float32) + p[0:1, :HID]
        mu = jnp.mean(h, axis=1, keepdims=True)
        var = jnp.mean((h - mu) ** 2, axis=1, keepdims=True)
        h = (h - mu) * lax.rsqrt(var + 1e-5) * p[1:2, :HID] + p[2:3, :HID]
        h = jnp.maximum(h, 0.0)
        h = jnp.dot(h, wlin_ref[...],
                    preferred_element_type=jnp.float32) + p[3:4, :OUT]
        h = jnp.maximum(jnp.dot(h, wq1_ref[...],
                                preferred_element_type=jnp.float32)
                        + p[4:5, :HDIM], 0.0)
        out_ref[...] = jnp.dot(h, wq2_ref[...],
                               preferred_element_type=jnp.float32) + p[5:6, :NA]

    q = pl.pallas_call(
        node_body,
        grid=(GN,),
        in_specs=[pl.BlockSpec((NW, TN, HID), lambda i: (0, i, 0)),
                  pl.BlockSpec((TN, IN), lambda i: (i, 0)),
                  pl.BlockSpec((IN, HID), lambda i: (0, 0)),
                  pl.BlockSpec((HID, OUT), lambda i: (0, 0)),
                  pl.BlockSpec((OUT, HDIM), lambda i: (0, 0)),
                  pl.BlockSpec((HDIM, NA), lambda i: (0, 0)),
                  pl.BlockSpec((8, 64), lambda i: (0, 0))],
        out_specs=pl.BlockSpec((TN, NA), lambda i: (i, 0)),
        out_shape=jax.ShapeDtypeStruct((N, NA), jnp.float32),
    )(aggall, x, root, Wlin.T, Wq1.T, Wq2.T, prm)
    return q


# speedup vs baseline: 2.1851x; 1.0237x over previous
import functools

import jax
import jax.numpy as jnp
from jax import lax
from jax.experimental import pallas as pl
from jax.experimental.pallas import tpu as pltpu
from jax.experimental.pallas import tpu_sc as plsc

NC = 2
NS = 16
NW = NC * NS
CH = 128
MSGW = 16


def kernel(x, edge_index, edge_attr, W1, b1, root, bconv, gamma, beta,
           Wlin, blin, Wq1, bq1, Wq2, bq2):
    N, IN = x.shape
    E, EDIM = edge_attr.shape
    HID = root.shape[1]
    OUT = Wlin.shape[0]
    HDIM = Wq1.shape[0]
    NA = Wq2.shape[0]

    src = edge_index[0]
    dst = edge_index[1]

    W1q = W1.reshape(IN, HID, EDIM).transpose(2, 1, 0).reshape(EDIM, HID * IN)
    b1q = jnp.broadcast_to(b1.reshape(IN, HID).T.reshape(1, HID * IN),
                           (8, HID * IN))

    EPW = E // NW
    NCHUNK = EPW // CH
    TAIL = EPW - NCHUNK * CH

    mesh = plsc.VectorSubcoreMesh(core_axis_name="c", subcore_axis_name="s",
                                  num_cores=NC, num_subcores=NS)

    @functools.partial(
        pl.kernel,
        out_type=jax.ShapeDtypeStruct((E, IN), jnp.float32),
        mesh=mesh,
        scratch_types=[pltpu.VMEM((CH,), jnp.int32),
                       pltpu.VMEM((CH,), jnp.int32),
                       pltpu.VMEM((CH, IN), jnp.float32),
                       pltpu.VMEM((CH, IN), jnp.float32),
                       pltpu.VMEM((max(TAIL, 8),), jnp.int32),
                       pltpu.VMEM((max(TAIL, 8), IN), jnp.float32),
                       pltpu.SemaphoreType.DMA,
                       pltpu.SemaphoreType.DMA,
                       pltpu.SemaphoreType.DMA,
                       pltpu.SemaphoreType.DMA],
    )
    def gather_k(x_hbm, src_hbm, out_hbm, idx_a, idx_b, rows_a, rows_b,
                 idx_t, rows_t, sem_ga, sem_gb, sem_wa, sem_wb):
        wid = lax.axis_index("c") * NS + lax.axis_index("s")
        base = wid * EPW
        NPAIR = NCHUNK // 2

        def pair_body(i, c):
            off_a = base + (2 * i) * CH
            off_b = off_a + CH
            pltpu.sync_copy(src_hbm.at[pl.ds(off_a, CH)], idx_a)
            ga = pltpu.async_copy(x_hbm.at[idx_a], rows_a, sem_ga)
            pltpu.sync_copy(src_hbm.at[pl.ds(off_b, CH)], idx_b)
            ga.wait()
            gb = pltpu.async_copy(x_hbm.at[idx_b], rows_b, sem_gb)
            wa = pltpu.async_copy(rows_a, out_hbm.at[pl.ds(off_a, CH)],
                                  sem_wa)
            gb.wait()
            wb = pltpu.async_copy(rows_b, out_hbm.at[pl.ds(off_b, CH)],
                                  sem_wb)
            wa.wait()
            wb.wait()
            return c

        lax.fori_loop(0, NPAIR, pair_body, 0)
        if NCHUNK % 2:
            off = base + (NCHUNK - 1) * CH
            pltpu.sync_copy(src_hbm.at[pl.ds(off, CH)], idx_a)
            pltpu.async_copy(x_hbm.at[idx_a], rows_a, sem_ga).wait()
            pltpu.sync_copy(rows_a, out_hbm.at[pl.ds(off, CH)])
        if TAIL:
            off = base + NCHUNK * CH
            pltpu.sync_copy(src_hbm.at[pl.ds(off, TAIL)], idx_t)
            pltpu.async_copy(x_hbm.at[idx_t], rows_t, sem_ga).wait()
            pltpu.sync_copy(rows_t, out_hbm.at[pl.ds(off, TAIL)])

    x_src = gather_k(x, src)

    TE = 2000
    GE = E // TE

    sel = (jnp.arange(HID * IN)[:, None] // IN ==
           jnp.arange(MSGW)[None, :]).astype(jnp.float32)

    def edge_body(ea_ref, xs_ref, w_ref, b_ref, sel_ref, out_ref):
        P = jnp.dot(ea_ref[...], w_ref[...],
                    preferred_element_type=jnp.float32) + b_ref[0:1, :]
        P = jnp.maximum(P, 0.0)
        xs8 = jnp.concatenate([xs_ref[...]] * HID, axis=1)
        out_ref[...] = jnp.dot(P * xs8, sel_ref[...],
                               preferred_element_type=jnp.float32)

    msg = pl.pallas_call(
        edge_body,
        grid=(GE,),
        in_specs=[pl.BlockSpec((TE, EDIM), lambda i: (i, 0)),
                  pl.BlockSpec((TE, IN), lambda i: (i, 0)),
                  pl.BlockSpec((EDIM, HID * IN), lambda i: (0, 0)),
                  pl.BlockSpec((8, HID * IN), lambda i: (0, 0)),
                  pl.BlockSpec((HID * IN, MSGW), lambda i: (0, 0))],
        out_specs=pl.BlockSpec((TE, MSGW), lambda i: (i, 0)),
        out_shape=jax.ShapeDtypeStruct((E, MSGW), jnp.float32),
    )(edge_attr, x_src, W1q, b1q, sel)

    NPAD = ((N + NS * 8 - 1) // (NS * 8)) * NS * 8
    SUBS = CH // 16

    @functools.partial(
        pl.kernel,
        out_type=jax.ShapeDtypeStruct((NW, NPAD * HID), jnp.float32),
        mesh=mesh,
        scratch_types=[pltpu.VMEM((CH,), jnp.int32),
                       pltpu.VMEM((CH * MSGW,), jnp.float32),
                       pltpu.VMEM((16,), jnp.int32),
                       pltpu.VMEM((16 * MSGW,), jnp.float32),
                       pltpu.VMEM((NPAD * HID,), jnp.float32)],
        compiler_params=pltpu.CompilerParams(needs_layout_passes=False),
    )
    def scatter_k(msg_hbm, dst_hbm, zero_hbm, out_hbm,
                  idx_v, rows_v, idx_t, rows_t, agg_v):
        wid = lax.axis_index("c") * NS + lax.axis_index("s")
        pltpu.sync_copy(zero_hbm, agg_v)
        base = wid * EPW
        lanes = lax.iota(jnp.int32, 16)

        def chunk_body(i, c):
            off = base + i * CH
            pltpu.sync_copy(dst_hbm.at[pl.ds(off, CH)], idx_v)
            pltpu.sync_copy(msg_hbm.at[pl.ds(off * MSGW, CH * MSGW)], rows_v)

            def sub_body(j, c2):
                rowbase = j * 16
                dsub = idx_v[pl.ds(rowbase, 16)]
                vidx = (rowbase + lanes) * MSGW
                for o in range(HID):
                    vals = plsc.load_gather(rows_v, [vidx + o])
                    plsc.addupdate_scatter(agg_v, [dsub * HID + o], vals)
                return c2

            lax.fori_loop(0, SUBS, sub_body, 0)
            return c

        lax.fori_loop(0, NCHUNK, chunk_body, 0)
        if TAIL:
            off = base + NCHUNK * CH
            idx_t[...] = jnp.zeros((16,), jnp.int32)
            pltpu.sync_copy(dst_hbm.at[pl.ds(off, TAIL)],
                            idx_t.at[pl.ds(0, TAIL)])
            pltpu.sync_copy(msg_hbm.at[pl.ds(off * MSGW, TAIL * MSGW)],
                            rows_t.at[pl.ds(0, TAIL * MSGW)])
            dsub = idx_t[...]
            tmask = lanes < TAIL
            for o in range(HID):
                vals = plsc.load_gather(rows_t, [lanes * MSGW + o])
                plsc.addupdate_scatter(agg_v, [dsub * HID + o], vals,
                                       mask=tmask)
        pltpu.sync_copy(agg_v, out_hbm.at[wid])

    aggall = scatter_k(msg.reshape(E * MSGW), dst,
                       jnp.zeros((NPAD * HID,), jnp.float32))
    aggall = aggall.reshape(NW, NPAD, HID)

    TN = 1000
    GN = N // TN
    prm = jnp.zeros((8, 64), jnp.float32)
    prm = (prm.at[0, :HID].set(bconv).at[1, :HID].set(gamma)
              .at[2, :HID].set(beta).at[3, :OUT].set(blin)
              .at[4, :HDIM].set(bq1).at[5, :NA].set(bq2))

    def node_body(ag_ref, x_ref, root_ref, wlin_ref, wq1_ref,
                  wq2_ref, prm_ref, out_ref):
        p = prm_ref[...]
        agg = jnp.sum(ag_ref[...], axis=0)
        h = agg + jnp.dot(x_ref[...], root_ref[...],
                          preferred_element_type=jnp.float32) + p[0:1, :HID]
        mu = jnp.mean(h, axis=1, keepdims=True)
        var = jnp.mean((h - mu) ** 2, axis=1, keepdims=True)
        h = (h - mu) * lax.rsqrt(var + 1e-5) * p[1:2, :HID] + p[2:3, :HID]
        h = jnp.maximum(h, 0.0)
        h = jnp.dot(h, wlin_ref[...],
                    preferred_element_type=jnp.float32) + p[3:4, :OUT]
        h = jnp.maximum(jnp.dot(h, wq1_ref[...],
                                preferred_element_type=jnp.float32)
                        + p[4:5, :HDIM], 0.0)
        out_ref[...] = jnp.dot(h, wq2_ref[...],
                               preferred_element_type=jnp.float32) + p[5:6, :NA]

    q = pl.pallas_call(
        node_body,
        grid=(GN,),
        in_specs=[pl.BlockSpec((NW, TN, HID), lambda i: (0, i, 0)),
                  pl.BlockSpec((TN, IN), lambda i: (i, 0)),
                  pl.BlockSpec((IN, HID), lambda i: (0, 0)),
                  pl.BlockSpec((HID, OUT), lambda i: (0, 0)),
                  pl.BlockSpec((OUT, HDIM), lambda i: (0, 0)),
                  pl.BlockSpec((HDIM, NA), lambda i: (0, 0)),
                  pl.BlockSpec((8, 64), lambda i: (0, 0))],
        out_specs=pl.BlockSpec((TN, NA), lambda i: (i, 0)),
        out_shape=jax.ShapeDtypeStruct((N, NA), jnp.float32),
    )(aggall, x, root, Wlin.T, Wq1.T, Wq2.T, prm)
    return q


# speedup vs baseline: 2.2912x; 1.0486x over previous
import functools

import jax
import jax.numpy as jnp
from jax import lax
from jax.experimental import pallas as pl
from jax.experimental.pallas import tpu as pltpu
from jax.experimental.pallas import tpu_sc as plsc

NC = 2
NS = 16
NW = NC * NS
CH = 128
MSGW = 16


def kernel(x, edge_index, edge_attr, W1, b1, root, bconv, gamma, beta,
           Wlin, blin, Wq1, bq1, Wq2, bq2):
    N, IN = x.shape
    E, EDIM = edge_attr.shape
    HID = root.shape[1]
    OUT = Wlin.shape[0]
    HDIM = Wq1.shape[0]
    NA = Wq2.shape[0]

    src = edge_index[0]
    dst = edge_index[1]

    W1q = W1.reshape(IN, HID, EDIM).transpose(2, 1, 0).reshape(EDIM, HID * IN)
    b1q = jnp.broadcast_to(b1.reshape(IN, HID).T.reshape(1, HID * IN),
                           (8, HID * IN))

    EPW = E // NW
    NCHUNK = EPW // CH
    TAIL = EPW - NCHUNK * CH

    mesh = plsc.VectorSubcoreMesh(core_axis_name="c", subcore_axis_name="s",
                                  num_cores=NC, num_subcores=NS)

    @functools.partial(
        pl.kernel,
        out_type=jax.ShapeDtypeStruct((E, IN), jnp.float32),
        mesh=mesh,
        scratch_types=[pltpu.VMEM((CH,), jnp.int32),
                       pltpu.VMEM((CH,), jnp.int32),
                       pltpu.VMEM((CH, IN), jnp.float32),
                       pltpu.VMEM((CH, IN), jnp.float32),
                       pltpu.VMEM((max(TAIL, 8),), jnp.int32),
                       pltpu.VMEM((max(TAIL, 8), IN), jnp.float32),
                       pltpu.SemaphoreType.DMA,
                       pltpu.SemaphoreType.DMA,
                       pltpu.SemaphoreType.DMA,
                       pltpu.SemaphoreType.DMA],
    )
    def gather_k(x_hbm, src_hbm, out_hbm, idx_a, idx_b, rows_a, rows_b,
                 idx_t, rows_t, sem_ga, sem_gb, sem_wa, sem_wb):
        wid = lax.axis_index("c") * NS + lax.axis_index("s")
        base = wid * EPW
        NPAIR = NCHUNK // 2

        def pair_body(i, c):
            off_a = base + (2 * i) * CH
            off_b = off_a + CH
            pltpu.sync_copy(src_hbm.at[pl.ds(off_a, CH)], idx_a)
            ga = pltpu.async_copy(x_hbm.at[idx_a], rows_a, sem_ga)
            pltpu.sync_copy(src_hbm.at[pl.ds(off_b, CH)], idx_b)
            ga.wait()
            gb = pltpu.async_copy(x_hbm.at[idx_b], rows_b, sem_gb)
            wa = pltpu.async_copy(rows_a, out_hbm.at[pl.ds(off_a, CH)],
                                  sem_wa)
            gb.wait()
            wb = pltpu.async_copy(rows_b, out_hbm.at[pl.ds(off_b, CH)],
                                  sem_wb)
            wa.wait()
            wb.wait()
            return c

        lax.fori_loop(0, NPAIR, pair_body, 0)
        if NCHUNK % 2:
            off = base + (NCHUNK - 1) * CH
            pltpu.sync_copy(src_hbm.at[pl.ds(off, CH)], idx_a)
            pltpu.async_copy(x_hbm.at[idx_a], rows_a, sem_ga).wait()
            pltpu.sync_copy(rows_a, out_hbm.at[pl.ds(off, CH)])
        if TAIL:
            off = base + NCHUNK * CH
            pltpu.sync_copy(src_hbm.at[pl.ds(off, TAIL)], idx_t)
            pltpu.async_copy(x_hbm.at[idx_t], rows_t, sem_ga).wait()
            pltpu.sync_copy(rows_t, out_hbm.at[pl.ds(off, TAIL)])

    x_src = gather_k(x, src)

    TE = 2000
    GE = E // TE

    sel = (jnp.arange(HID * IN)[:, None] // IN ==
           jnp.arange(MSGW)[None, :]).astype(jnp.float32)

    def edge_body(ea_ref, xs_ref, w_ref, b_ref, sel_ref, out_ref):
        P = jnp.dot(ea_ref[...], w_ref[...],
                    preferred_element_type=jnp.float32) + b_ref[0:1, :]
        P = jnp.maximum(P, 0.0)
        xs8 = jnp.concatenate([xs_ref[...]] * HID, axis=1)
        out_ref[...] = jnp.dot(P * xs8, sel_ref[...],
                               preferred_element_type=jnp.float32)

    msg = pl.pallas_call(
        edge_body,
        grid=(GE,),
        in_specs=[pl.BlockSpec((TE, EDIM), lambda i: (i, 0)),
                  pl.BlockSpec((TE, IN), lambda i: (i, 0)),
                  pl.BlockSpec((EDIM, HID * IN), lambda i: (0, 0)),
                  pl.BlockSpec((8, HID * IN), lambda i: (0, 0)),
                  pl.BlockSpec((HID * IN, MSGW), lambda i: (0, 0))],
        out_specs=pl.BlockSpec((TE, MSGW), lambda i: (i, 0)),
        out_shape=jax.ShapeDtypeStruct((E, MSGW), jnp.float32),
    )(edge_attr, x_src, W1q, b1q, sel)

    NPAD = ((N + NS * 8 - 1) // (NS * 8)) * NS * 8
    SUBS = CH // 16

    @functools.partial(
        pl.kernel,
        out_type=jax.ShapeDtypeStruct((NW, NPAD * HID), jnp.float32),
        mesh=mesh,
        scratch_types=[pltpu.VMEM((CH,), jnp.int32),
                       pltpu.VMEM((CH,), jnp.int32),
                       pltpu.VMEM((CH * MSGW,), jnp.float32),
                       pltpu.VMEM((CH * MSGW,), jnp.float32),
                       pltpu.VMEM((16,), jnp.int32),
                       pltpu.VMEM((16 * MSGW,), jnp.float32),
                       pltpu.VMEM((NPAD * HID,), jnp.float32),
                       pltpu.SemaphoreType.DMA,
                       pltpu.SemaphoreType.DMA,
                       pltpu.SemaphoreType.DMA,
                       pltpu.SemaphoreType.DMA],
        compiler_params=pltpu.CompilerParams(needs_layout_passes=False),
    )
    def scatter_k(msg_hbm, dst_hbm, zero_hbm, out_hbm,
                  idx_a, idx_b, rows_a, rows_b, idx_t, rows_t, agg_v,
                  sem_ia, sem_ra, sem_ib, sem_rb):
        wid = lax.axis_index("c") * NS + lax.axis_index("s")
        pltpu.sync_copy(zero_hbm, agg_v)
        base = wid * EPW
        lanes = lax.iota(jnp.int32, 16)

        def accumulate(idx_v, rows_v):
            def sub_body(j, c2):
                rowbase = j * 16
                dsub = idx_v[pl.ds(rowbase, 16)]
                vidx = (rowbase + lanes) * MSGW
                for o in range(HID):
                    vals = plsc.load_gather(rows_v, [vidx + o])
                    plsc.addupdate_scatter(agg_v, [dsub * HID + o], vals)
                return c2

            lax.fori_loop(0, SUBS, sub_body, 0)

        NPAIR2 = NCHUNK // 2

        def pair_body(i, c):
            off_a = base + (2 * i) * CH
            off_b = off_a + CH
            ia = pltpu.async_copy(dst_hbm.at[pl.ds(off_a, CH)], idx_a,
                                  sem_ia)
            ra = pltpu.async_copy(
                msg_hbm.at[pl.ds(off_a * MSGW, CH * MSGW)], rows_a, sem_ra)
            ib = pltpu.async_copy(dst_hbm.at[pl.ds(off_b, CH)], idx_b,
                                  sem_ib)
            rb = pltpu.async_copy(
                msg_hbm.at[pl.ds(off_b * MSGW, CH * MSGW)], rows_b, sem_rb)
            ia.wait()
            ra.wait()
            accumulate(idx_a, rows_a)
            ib.wait()
            rb.wait()
            accumulate(idx_b, rows_b)
            return c

        lax.fori_loop(0, NPAIR2, pair_body, 0)
        if NCHUNK % 2:
            off = base + (NCHUNK - 1) * CH
            pltpu.sync_copy(dst_hbm.at[pl.ds(off, CH)], idx_a)
            pltpu.sync_copy(msg_hbm.at[pl.ds(off * MSGW, CH * MSGW)], rows_a)
            accumulate(idx_a, rows_a)
        if TAIL:
            off = base + NCHUNK * CH
            idx_t[...] = jnp.zeros((16,), jnp.int32)
            pltpu.sync_copy(dst_hbm.at[pl.ds(off, TAIL)],
                            idx_t.at[pl.ds(0, TAIL)])
            pltpu.sync_copy(msg_hbm.at[pl.ds(off * MSGW, TAIL * MSGW)],
                            rows_t.at[pl.ds(0, TAIL * MSGW)])
            dsub = idx_t[...]
            tmask = lanes < TAIL
            for o in range(HID):
                vals = plsc.load_gather(rows_t, [lanes * MSGW + o])
                plsc.addupdate_scatter(agg_v, [dsub * HID + o], vals,
                                       mask=tmask)
        pltpu.sync_copy(agg_v, out_hbm.at[wid])

    aggall = scatter_k(msg.reshape(E * MSGW), dst,
                       jnp.zeros((NPAD * HID,), jnp.float32))
    aggall = aggall.reshape(NW, NPAD, HID)

    TN = 1000
    GN = N // TN
    prm = jnp.zeros((8, 64), jnp.float32)
    prm = (prm.at[0, :HID].set(bconv).at[1, :HID].set(gamma)
              .at[2, :HID].set(beta).at[3, :OUT].set(blin)
              .at[4, :HDIM].set(bq1).at[5, :NA].set(bq2))

    def node_body(ag_ref, x_ref, root_ref, wlin_ref, wq1_ref,
                  wq2_ref, prm_ref, out_ref):
        p = prm_ref[...]
        agg = jnp.sum(ag_ref[...], axis=0)
        h = agg + jnp.dot(x_ref[...], root_ref[...],
                          preferred_element_type=jnp.float32) + p[0:1, :HID]
        mu = jnp.mean(h, axis=1, keepdims=True)
        var = jnp.mean((h - mu) ** 2, axis=1, keepdims=True)
        h = (h - mu) * lax.rsqrt(var + 1e-5) * p[1:2, :HID] + p[2:3, :HID]
        h = jnp.maximum(h, 0.0)
        h = jnp.dot(h, wlin_ref[...],
                    preferred_element_type=jnp.float32) + p[3:4, :OUT]
        h = jnp.maximum(jnp.dot(h, wq1_ref[...],
                                preferred_element_type=jnp.float32)
                        + p[4:5, :HDIM], 0.0)
        out_ref[...] = jnp.dot(h, wq2_ref[...],
                               preferred_element_type=jnp.float32) + p[5:6, :NA]

    q = pl.pallas_call(
        node_body,
        grid=(GN,),
        in_specs=[pl.BlockSpec((NW, TN, HID), lambda i: (0, i, 0)),
                  pl.BlockSpec((TN, IN), lambda i: (i, 0)),
                  pl.BlockSpec((IN, HID), lambda i: (0, 0)),
                  pl.BlockSpec((HID, OUT), lambda i: (0, 0)),
                  pl.BlockSpec((OUT, HDIM), lambda i: (0, 0)),
                  pl.BlockSpec((HDIM, NA), lambda i: (0, 0)),
                  pl.BlockSpec((8, 64), lambda i: (0, 0))],
        out_specs=pl.BlockSpec((TN, NA), lambda i: (i, 0)),
        out_shape=jax.ShapeDtypeStruct((N, NA), jnp.float32),
    )(aggall, x, root, Wlin.T, Wq1.T, Wq2.T, prm)
    return q


# speedup vs baseline: 2.2939x; 1.0012x over previous
import functools

import jax
import jax.numpy as jnp
from jax import lax
from jax.experimental import pallas as pl
from jax.experimental.pallas import tpu as pltpu
from jax.experimental.pallas import tpu_sc as plsc

NC = 2
NS = 16
NW = NC * NS
CH = 128
MSGW = 16


def kernel(x, edge_index, edge_attr, W1, b1, root, bconv, gamma, beta,
           Wlin, blin, Wq1, bq1, Wq2, bq2):
    N, IN = x.shape
    E, EDIM = edge_attr.shape
    HID = root.shape[1]
    OUT = Wlin.shape[0]
    HDIM = Wq1.shape[0]
    NA = Wq2.shape[0]

    src = edge_index[0]
    dst = edge_index[1]

    W1q = W1.reshape(IN, HID, EDIM).transpose(2, 1, 0).reshape(EDIM, HID * IN)
    b1q = b1.reshape(IN, HID).T.reshape(1, HID * IN)
    Wk = jnp.concatenate([W1q, b1q], axis=0)

    EPW = E // NW
    NCHUNK = EPW // CH
    TAIL = EPW - NCHUNK * CH

    mesh = plsc.VectorSubcoreMesh(core_axis_name="c", subcore_axis_name="s",
                                  num_cores=NC, num_subcores=NS)

    @functools.partial(
        pl.kernel,
        out_type=jax.ShapeDtypeStruct((E, IN), jnp.float32),
        mesh=mesh,
        scratch_types=[pltpu.VMEM((CH,), jnp.int32),
                       pltpu.VMEM((CH,), jnp.int32),
                       pltpu.VMEM((CH, IN), jnp.float32),
                       pltpu.VMEM((CH, IN), jnp.float32),
                       pltpu.VMEM((max(TAIL, 8),), jnp.int32),
                       pltpu.VMEM((max(TAIL, 8), IN), jnp.float32),
                       pltpu.SemaphoreType.DMA,
                       pltpu.SemaphoreType.DMA,
                       pltpu.SemaphoreType.DMA,
                       pltpu.SemaphoreType.DMA],
    )
    def gather_k(x_hbm, src_hbm, out_hbm, idx_a, idx_b, rows_a, rows_b,
                 idx_t, rows_t, sem_ga, sem_gb, sem_wa, sem_wb):
        wid = lax.axis_index("c") * NS + lax.axis_index("s")
        base = wid * EPW
        NPAIR = NCHUNK // 2

        def pair_body(i, c):
            off_a = base + (2 * i) * CH
            off_b = off_a + CH
            pltpu.sync_copy(src_hbm.at[pl.ds(off_a, CH)], idx_a)
            ga = pltpu.async_copy(x_hbm.at[idx_a], rows_a, sem_ga)
            pltpu.sync_copy(src_hbm.at[pl.ds(off_b, CH)], idx_b)
            ga.wait()
            gb = pltpu.async_copy(x_hbm.at[idx_b], rows_b, sem_gb)
            wa = pltpu.async_copy(rows_a, out_hbm.at[pl.ds(off_a, CH)],
                                  sem_wa)
            gb.wait()
            wb = pltpu.async_copy(rows_b, out_hbm.at[pl.ds(off_b, CH)],
                                  sem_wb)
            wa.wait()
            wb.wait()
            return c

        lax.fori_loop(0, NPAIR, pair_body, 0)
        if NCHUNK % 2:
            off = base + (NCHUNK - 1) * CH
            pltpu.sync_copy(src_hbm.at[pl.ds(off, CH)], idx_a)
            pltpu.async_copy(x_hbm.at[idx_a], rows_a, sem_ga).wait()
            pltpu.sync_copy(rows_a, out_hbm.at[pl.ds(off, CH)])
        if TAIL:
            off = base + NCHUNK * CH
            pltpu.sync_copy(src_hbm.at[pl.ds(off, TAIL)], idx_t)
            pltpu.async_copy(x_hbm.at[idx_t], rows_t, sem_ga).wait()
            pltpu.sync_copy(rows_t, out_hbm.at[pl.ds(off, TAIL)])

    x_src = gather_k(x, src)

    TE = 2000
    GE = E // TE

    sel = (jnp.arange(HID * IN)[:, None] // IN ==
           jnp.arange(MSGW)[None, :]).astype(jnp.float32)

    def edge_body(ea_ref, xs_ref, w_ref, sel_ref, out_ref):
        ea1 = jnp.concatenate(
            [ea_ref[...], jnp.ones((TE, 1), jnp.float32)], axis=1)
        P = jnp.dot(ea1, w_ref[...], preferred_element_type=jnp.float32)
        P = jnp.maximum(P, 0.0)
        xs8 = jnp.concatenate([xs_ref[...]] * HID, axis=1)
        out_ref[...] = jnp.dot(P * xs8, sel_ref[...],
                               preferred_element_type=jnp.float32)

    msg = pl.pallas_call(
        edge_body,
        grid=(GE,),
        in_specs=[pl.BlockSpec((TE, EDIM), lambda i: (i, 0)),
                  pl.BlockSpec((TE, IN), lambda i: (i, 0)),
                  pl.BlockSpec((EDIM + 1, HID * IN), lambda i: (0, 0)),
                  pl.BlockSpec((HID * IN, MSGW), lambda i: (0, 0))],
        out_specs=pl.BlockSpec((TE, MSGW), lambda i: (i, 0)),
        out_shape=jax.ShapeDtypeStruct((E, MSGW), jnp.float32),
    )(edge_attr, x_src, Wk, sel)

    NPAD = ((N + NS * 8 - 1) // (NS * 8)) * NS * 8
    SUBS = CH // 16

    @functools.partial(
        pl.kernel,
        out_type=jax.ShapeDtypeStruct((NW, NPAD * HID), jnp.float32),
        mesh=mesh,
        scratch_types=[pltpu.VMEM((CH,), jnp.int32),
                       pltpu.VMEM((CH,), jnp.int32),
                       pltpu.VMEM((CH * MSGW,), jnp.float32),
                       pltpu.VMEM((CH * MSGW,), jnp.float32),
                       pltpu.VMEM((16,), jnp.int32),
                       pltpu.VMEM((16 * MSGW,), jnp.float32),
                       pltpu.VMEM((NPAD * HID,), jnp.float32),
                       pltpu.SemaphoreType.DMA,
                       pltpu.SemaphoreType.DMA,
                       pltpu.SemaphoreType.DMA,
                       pltpu.SemaphoreType.DMA],
        compiler_params=pltpu.CompilerParams(needs_layout_passes=False),
    )
    def scatter_k(msg_hbm, dst_hbm, zero_hbm, out_hbm,
                  idx_a, idx_b, rows_a, rows_b, idx_t, rows_t, agg_v,
                  sem_ia, sem_ra, sem_ib, sem_rb):
        wid = lax.axis_index("c") * NS + lax.axis_index("s")
        pltpu.sync_copy(zero_hbm, agg_v)
        base = wid * EPW
        lanes = lax.iota(jnp.int32, 16)

        def accumulate(idx_v, rows_v):
            def sub_body(j, c2):
                rowbase = j * 16
                dsub = idx_v[pl.ds(rowbase, 16)]
                vidx = (rowbase + lanes) * MSGW
                for o in range(HID):
                    vals = plsc.load_gather(rows_v, [vidx + o])
                    plsc.addupdate_scatter(agg_v, [dsub * HID + o], vals)
                return c2

            lax.fori_loop(0, SUBS, sub_body, 0)

        NPAIR2 = NCHUNK // 2

        def pair_body(i, c):
            off_a = base + (2 * i) * CH
            off_b = off_a + CH
            ia = pltpu.async_copy(dst_hbm.at[pl.ds(off_a, CH)], idx_a,
                                  sem_ia)
            ra = pltpu.async_copy(
                msg_hbm.at[pl.ds(off_a * MSGW, CH * MSGW)], rows_a, sem_ra)
            ib = pltpu.async_copy(dst_hbm.at[pl.ds(off_b, CH)], idx_b,
                                  sem_ib)
            rb = pltpu.async_copy(
                msg_hbm.at[pl.ds(off_b * MSGW, CH * MSGW)], rows_b, sem_rb)
            ia.wait()
            ra.wait()
            accumulate(idx_a, rows_a)
            ib.wait()
            rb.wait()
            accumulate(idx_b, rows_b)
            return c

        lax.fori_loop(0, NPAIR2, pair_body, 0)
        if NCHUNK % 2:
            off = base + (NCHUNK - 1) * CH
            pltpu.sync_copy(dst_hbm.at[pl.ds(off, CH)], idx_a)
            pltpu.sync_copy(msg_hbm.at[pl.ds(off * MSGW, CH * MSGW)], rows_a)
            accumulate(idx_a, rows_a)
        if TAIL:
            off = base + NCHUNK * CH
            idx_t[...] = jnp.zeros((16,), jnp.int32)
            pltpu.sync_copy(dst_hbm.at[pl.ds(off, TAIL)],
                            idx_t.at[pl.ds(0, TAIL)])
            pltpu.sync_copy(msg_hbm.at[pl.ds(off * MSGW, TAIL * MSGW)],
                            rows_t.at[pl.ds(0, TAIL * MSGW)])
            dsub = idx_t[...]
            tmask = lanes < TAIL
            for o in range(HID):
                vals = plsc.load_gather(rows_t, [lanes * MSGW + o])
                plsc.addupdate_scatter(agg_v, [dsub * HID + o], vals,
                                       mask=tmask)
        pltpu.sync_copy(agg_v, out_hbm.at[wid])

    aggall = scatter_k(msg.reshape(E * MSGW), dst,
                       jnp.zeros((NPAD * HID,), jnp.float32))
    aggall = aggall.reshape(NW, NPAD, HID)

    TN = 1000
    GN = N // TN
    prm = jnp.zeros((8, 64), jnp.float32)
    prm = (prm.at[0, :HID].set(bconv).at[1, :HID].set(gamma)
              .at[2, :HID].set(beta).at[3, :OUT].set(blin)
              .at[4, :HDIM].set(bq1).at[5, :NA].set(bq2))

    def node_body(ag_ref, x_ref, root_ref, wlin_ref, wq1_ref,
                  wq2_ref, prm_ref, out_ref):
        p = prm_ref[...]
        agg = jnp.sum(ag_ref[...], axis=0)
        h = agg + jnp.dot(x_ref[...], root_ref[...],
                          preferred_element_type=jnp.float32) + p[0:1, :HID]
        mu = jnp.mean(h, axis=1, keepdims=True)
        var = jnp.mean((h - mu) ** 2, axis=1, keepdims=True)
        h = (h - mu) * lax.rsqrt(var + 1e-5) * p[1:2, :HID] + p[2:3, :HID]
        h = jnp.maximum(h, 0.0)
        h = jnp.dot(h, wlin_ref[...],
                    preferred_element_type=jnp.float32) + p[3:4, :OUT]
        h = jnp.maximum(jnp.dot(h, wq1_ref[...],
                                preferred_element_type=jnp.float32)
                        + p[4:5, :HDIM], 0.0)
        out_ref[...] = jnp.dot(h, wq2_ref[...],
                               preferred_element_type=jnp.float32) + p[5:6, :NA]

    q = pl.pallas_call(
        node_body,
        grid=(GN,),
        in_specs=[pl.BlockSpec((NW, TN, HID), lambda i: (0, i, 0)),
                  pl.BlockSpec((TN, IN), lambda i: (i, 0)),
                  pl.BlockSpec((IN, HID), lambda i: (0, 0)),
                  pl.BlockSpec((HID, OUT), lambda i: (0, 0)),
                  pl.BlockSpec((OUT, HDIM), lambda i: (0, 0)),
                  pl.BlockSpec((HDIM, NA), lambda i: (0, 0)),
                  pl.BlockSpec((8, 64), lambda i: (0, 0))],
        out_specs=pl.BlockSpec((TN, NA), lambda i: (i, 0)),
        out_shape=jax.ShapeDtypeStruct((N, NA), jnp.float32),
    )(aggall, x, root, Wlin.T, Wq1.T, Wq2.T, prm)
    return q


# speedup vs baseline: 2.3280x; 1.0149x over previous
import functools

import jax
import jax.numpy as jnp
from jax import lax
from jax.experimental import pallas as pl
from jax.experimental.pallas import tpu as pltpu
from jax.experimental.pallas import tpu_sc as plsc

NC = 2
NS = 16
NW = NC * NS
CH = 128
MSGW = 16


def kernel(x, edge_index, edge_attr, W1, b1, root, bconv, gamma, beta,
           Wlin, blin, Wq1, bq1, Wq2, bq2):
    N, IN = x.shape
    E, EDIM = edge_attr.shape
    HID = root.shape[1]
    OUT = Wlin.shape[0]
    HDIM = Wq1.shape[0]
    NA = Wq2.shape[0]

    src = edge_index[0]
    dst = edge_index[1]

    W1q = W1.reshape(IN, HID, EDIM).transpose(2, 1, 0).reshape(EDIM, HID * IN)
    b1q = b1.reshape(IN, HID).T.reshape(1, HID * IN)
    Wk = jnp.concatenate([W1q, b1q], axis=0)

    EPW = E // NW
    NCHUNK = EPW // CH
    TAIL = EPW - NCHUNK * CH

    mesh = plsc.VectorSubcoreMesh(core_axis_name="c", subcore_axis_name="s",
                                  num_cores=NC, num_subcores=NS)

    @functools.partial(
        pl.kernel,
        out_type=jax.ShapeDtypeStruct((E, IN), jnp.float32),
        mesh=mesh,
        scratch_types=[pltpu.VMEM((CH,), jnp.int32),
                       pltpu.VMEM((CH,), jnp.int32),
                       pltpu.VMEM((CH, IN), jnp.float32),
                       pltpu.VMEM((CH, IN), jnp.float32),
                       pltpu.VMEM((max(TAIL, 8),), jnp.int32),
                       pltpu.VMEM((max(TAIL, 8), IN), jnp.float32),
                       pltpu.SemaphoreType.DMA,
                       pltpu.SemaphoreType.DMA,
                       pltpu.SemaphoreType.DMA,
                       pltpu.SemaphoreType.DMA],
    )
    def gather_k(x_hbm, src_hbm, out_hbm, idx_a, idx_b, rows_a, rows_b,
                 idx_t, rows_t, sem_ga, sem_gb, sem_wa, sem_wb):
        wid = lax.axis_index("c") * NS + lax.axis_index("s")
        base = wid * EPW
        NPAIR = NCHUNK // 2

        def pair_body(i, c):
            off_a = base + (2 * i) * CH
            off_b = off_a + CH
            pltpu.sync_copy(src_hbm.at[pl.ds(off_a, CH)], idx_a)
            ga = pltpu.async_copy(x_hbm.at[idx_a], rows_a, sem_ga)
            pltpu.sync_copy(src_hbm.at[pl.ds(off_b, CH)], idx_b)
            ga.wait()
            gb = pltpu.async_copy(x_hbm.at[idx_b], rows_b, sem_gb)
            wa = pltpu.async_copy(rows_a, out_hbm.at[pl.ds(off_a, CH)],
                                  sem_wa)
            gb.wait()
            wb = pltpu.async_copy(rows_b, out_hbm.at[pl.ds(off_b, CH)],
                                  sem_wb)
            wa.wait()
            wb.wait()
            return c

        lax.fori_loop(0, NPAIR, pair_body, 0)
        if NCHUNK % 2:
            off = base + (NCHUNK - 1) * CH
            pltpu.sync_copy(src_hbm.at[pl.ds(off, CH)], idx_a)
            pltpu.async_copy(x_hbm.at[idx_a], rows_a, sem_ga).wait()
            pltpu.sync_copy(rows_a, out_hbm.at[pl.ds(off, CH)])
        if TAIL:
            off = base + NCHUNK * CH
            pltpu.sync_copy(src_hbm.at[pl.ds(off, TAIL)], idx_t)
            pltpu.async_copy(x_hbm.at[idx_t], rows_t, sem_ga).wait()
            pltpu.sync_copy(rows_t, out_hbm.at[pl.ds(off, TAIL)])

    x_src = gather_k(x, src)

    TE = 3200
    GE = E // TE

    sel = (jnp.arange(HID * IN)[:, None] // IN ==
           jnp.arange(MSGW)[None, :]).astype(jnp.float32)

    def edge_body(ea_ref, xs_ref, w_ref, sel_ref, out_ref):
        ea1 = jnp.concatenate(
            [ea_ref[...], jnp.ones((TE, 1), jnp.float32)], axis=1)
        P = jnp.dot(ea1, w_ref[...], preferred_element_type=jnp.float32)
        P = jnp.maximum(P, 0.0)
        xs8 = jnp.concatenate([xs_ref[...]] * HID, axis=1)
        out_ref[...] = jnp.dot(P * xs8, sel_ref[...],
                               preferred_element_type=jnp.float32)

    msg = pl.pallas_call(
        edge_body,
        grid=(GE,),
        in_specs=[pl.BlockSpec((TE, EDIM), lambda i: (i, 0)),
                  pl.BlockSpec((TE, IN), lambda i: (i, 0)),
                  pl.BlockSpec((EDIM + 1, HID * IN), lambda i: (0, 0)),
                  pl.BlockSpec((HID * IN, MSGW), lambda i: (0, 0))],
        out_specs=pl.BlockSpec((TE, MSGW), lambda i: (i, 0)),
        out_shape=jax.ShapeDtypeStruct((E, MSGW), jnp.float32),
    )(edge_attr, x_src, Wk, sel)

    NPAD = ((N + NS * 8 - 1) // (NS * 8)) * NS * 8
    SUBS = CH // 16

    @functools.partial(
        pl.kernel,
        out_type=jax.ShapeDtypeStruct((NW, NPAD * HID), jnp.float32),
        mesh=mesh,
        scratch_types=[pltpu.VMEM((CH,), jnp.int32),
                       pltpu.VMEM((CH,), jnp.int32),
                       pltpu.VMEM((CH * MSGW,), jnp.float32),
                       pltpu.VMEM((CH * MSGW,), jnp.float32),
                       pltpu.VMEM((16,), jnp.int32),
                       pltpu.VMEM((16 * MSGW,), jnp.float32),
                       pltpu.VMEM((NPAD * HID,), jnp.float32),
                       pltpu.SemaphoreType.DMA,
                       pltpu.SemaphoreType.DMA,
                       pltpu.SemaphoreType.DMA,
                       pltpu.SemaphoreType.DMA],
        compiler_params=pltpu.CompilerParams(needs_layout_passes=False),
    )
    def scatter_k(msg_hbm, dst_hbm, zero_hbm, out_hbm,
                  idx_a, idx_b, rows_a, rows_b, idx_t, rows_t, agg_v,
                  sem_ia, sem_ra, sem_ib, sem_rb):
        wid = lax.axis_index("c") * NS + lax.axis_index("s")
        pltpu.sync_copy(zero_hbm, agg_v)
        base = wid * EPW
        lanes = lax.iota(jnp.int32, 16)

        def accumulate(idx_v, rows_v):
            def sub_body(j, c2):
                rowbase = j * 16
                dsub = idx_v[pl.ds(rowbase, 16)]
                vidx = (rowbase + lanes) * MSGW
                for o in range(HID):
                    vals = plsc.load_gather(rows_v, [vidx + o])
                    plsc.addupdate_scatter(agg_v, [dsub * HID + o], vals)
                return c2

            lax.fori_loop(0, SUBS, sub_body, 0)

        NPAIR2 = NCHUNK // 2

        def pair_body(i, c):
            off_a = base + (2 * i) * CH
            off_b = off_a + CH
            ia = pltpu.async_copy(dst_hbm.at[pl.ds(off_a, CH)], idx_a,
                                  sem_ia)
            ra = pltpu.async_copy(
                msg_hbm.at[pl.ds(off_a * MSGW, CH * MSGW)], rows_a, sem_ra)
            ib = pltpu.async_copy(dst_hbm.at[pl.ds(off_b, CH)], idx_b,
                                  sem_ib)
            rb = pltpu.async_copy(
                msg_hbm.at[pl.ds(off_b * MSGW, CH * MSGW)], rows_b, sem_rb)
            ia.wait()
            ra.wait()
            accumulate(idx_a, rows_a)
            ib.wait()
            rb.wait()
            accumulate(idx_b, rows_b)
            return c

        lax.fori_loop(0, NPAIR2, pair_body, 0)
        if NCHUNK % 2:
            off = base + (NCHUNK - 1) * CH
            pltpu.sync_copy(dst_hbm.at[pl.ds(off, CH)], idx_a)
            pltpu.sync_copy(msg_hbm.at[pl.ds(off * MSGW, CH * MSGW)], rows_a)
            accumulate(idx_a, rows_a)
        if TAIL:
            off = base + NCHUNK * CH
            idx_t[...] = jnp.zeros((16,), jnp.int32)
            pltpu.sync_copy(dst_hbm.at[pl.ds(off, TAIL)],
                            idx_t.at[pl.ds(0, TAIL)])
            pltpu.sync_copy(msg_hbm.at[pl.ds(off * MSGW, TAIL * MSGW)],
                            rows_t.at[pl.ds(0, TAIL * MSGW)])
            dsub = idx_t[...]
            tmask = lanes < TAIL
            for o in range(HID):
                vals = plsc.load_gather(rows_t, [lanes * MSGW + o])
                plsc.addupdate_scatter(agg_v, [dsub * HID + o], vals,
                                       mask=tmask)
        pltpu.sync_copy(agg_v, out_hbm.at[wid])

    aggall = scatter_k(msg.reshape(E * MSGW), dst,
                       jnp.zeros((NPAD * HID,), jnp.float32))
    aggall = aggall.reshape(NW, NPAD, HID)

    TN = 1000
    GN = N // TN
    prm = jnp.zeros((8, 64), jnp.float32)
    prm = (prm.at[0, :HID].set(bconv).at[1, :HID].set(gamma)
              .at[2, :HID].set(beta).at[3, :OUT].set(blin)
              .at[4, :HDIM].set(bq1).at[5, :NA].set(bq2))

    def node_body(ag_ref, x_ref, root_ref, wlin_ref, wq1_ref,
                  wq2_ref, prm_ref, out_ref):
        p = prm_ref[...]
        agg = jnp.sum(ag_ref[...], axis=0)
        h = agg + jnp.dot(x_ref[...], root_ref[...],
                          preferred_element_type=jnp.float32) + p[0:1, :HID]
        mu = jnp.mean(h, axis=1, keepdims=True)
        var = jnp.mean((h - mu) ** 2, axis=1, keepdims=True)
        h = (h - mu) * lax.rsqrt(var + 1e-5) * p[1:2, :HID] + p[2:3, :HID]
        h = jnp.maximum(h, 0.0)
        h = jnp.dot(h, wlin_ref[...],
                    preferred_element_type=jnp.float32) + p[3:4, :OUT]
        h = jnp.maximum(jnp.dot(h, wq1_ref[...],
                                preferred_element_type=jnp.float32)
                        + p[4:5, :HDIM], 0.0)
        out_ref[...] = jnp.dot(h, wq2_ref[...],
                               preferred_element_type=jnp.float32) + p[5:6, :NA]

    q = pl.pallas_call(
        node_body,
        grid=(GN,),
        in_specs=[pl.BlockSpec((NW, TN, HID), lambda i: (0, i, 0)),
                  pl.BlockSpec((TN, IN), lambda i: (i, 0)),
                  pl.BlockSpec((IN, HID), lambda i: (0, 0)),
                  pl.BlockSpec((HID, OUT), lambda i: (0, 0)),
                  pl.BlockSpec((OUT, HDIM), lambda i: (0, 0)),
                  pl.BlockSpec((HDIM, NA), lambda i: (0, 0)),
                  pl.BlockSpec((8, 64), lambda i: (0, 0))],
        out_specs=pl.BlockSpec((TN, NA), lambda i: (i, 0)),
        out_shape=jax.ShapeDtypeStruct((N, NA), jnp.float32),
    )(aggall, x, root, Wlin.T, Wq1.T, Wq2.T, prm)
    return q
